# Initial kernel scaffold; baseline (speedup 1.0000x reference)
import functools

import jax
import jax.numpy as jnp
from jax import lax
from jax.experimental import pallas as pl
from jax.experimental.pallas import tpu as pltpu
from jax.experimental.pallas import tpu_sc as plsc

N = 10000
E = 320000
D = 128
D_EDGE = 16
HIDDEN = 32

CH = 128
NCH = E // CH
EB = 4000
NB = 1000

F32 = jnp.float32


def _alpha_body(ea_ref, w1_ref, b1_ref, w2t_ref, b2_ref, out_ref):
    ea = ea_ref[...]
    h1 = jnp.dot(ea, w1_ref[...], preferred_element_type=F32) + b1_ref[...]
    h1 = h1 * jax.nn.sigmoid(h1)
    z = jnp.sum(h1 * w2t_ref[...], axis=1, keepdims=True) + b2_ref[...]
    a = jax.nn.sigmoid(z)
    out_ref[...] = jnp.broadcast_to(a, (EB, 16))


def _alpha16(edge_attr, W1, b1, W2, b2):
    return pl.pallas_call(
        _alpha_body,
        grid=(E // EB,),
        in_specs=[
            pl.BlockSpec((EB, D_EDGE), lambda i: (i, 0)),
            pl.BlockSpec((D_EDGE, HIDDEN), lambda i: (0, 0)),
            pl.BlockSpec((1, HIDDEN), lambda i: (0, 0)),
            pl.BlockSpec((1, HIDDEN), lambda i: (0, 0)),
            pl.BlockSpec((1, 1), lambda i: (0, 0)),
        ],
        out_specs=pl.BlockSpec((EB, 16), lambda i: (i, 0)),
        out_shape=jax.ShapeDtypeStruct((E, 16), F32),
    )(edge_attr, W1, b1.reshape(1, HIDDEN), W2.reshape(1, HIDDEN),
      b2.reshape(1, 1))


def _deg_call(nc, ns):
    nw = nc * ns
    rps = N // ns

    def body(a16_hbm, col_hbm, zer_hbm, out_hbm, col_v, a16_v, deg_sh):
        cid = lax.axis_index("c")
        sid = lax.axis_index("s")
        wid = sid * nc + cid
        pltpu.sync_copy(zer_hbm, deg_sh.at[pl.ds(sid * rps, rps)])
        plsc.subcore_barrier()

        @pl.loop(wid, NCH, step=nw)
        def _(c):
            base = c * CH
            pltpu.sync_copy(col_hbm.at[pl.ds(base, CH)], col_v)
            pltpu.sync_copy(a16_hbm.at[pl.ds(base, CH)], a16_v)
            pltpu.sync_copy(a16_v, deg_sh.at[col_v], add=True)

        plsc.subcore_barrier()
        pltpu.sync_copy(deg_sh.at[pl.ds(sid * rps, rps)],
                        out_hbm.at[cid, pl.ds(sid * rps, rps)])

    mesh = plsc.VectorSubcoreMesh(core_axis_name="c", subcore_axis_name="s")
    return pl.kernel(
        body,
        out_type=jax.ShapeDtypeStruct((nc, N, 16), F32),
        mesh=mesh,
        scratch_types=[
            pltpu.VMEM((CH,), jnp.int32),
            pltpu.VMEM((CH, 16), F32),
            pltpu.VMEM_SHARED((N, 16), F32),
        ],
    )


def _y_body(x_ref, w_ref, dp_ref, y_ref, dg_ref):
    xw = jnp.dot(x_ref[...], w_ref[...], preferred_element_type=F32)
    deg = jnp.sum(dp_ref[...], axis=0)
    dis = jnp.where(deg > 0, lax.rsqrt(deg), 0.0)
    y_ref[...] = xw * dis[:, 0:1]
    dg_ref[...] = deg


def _y_call(nc, x, W, degparts):
    return pl.pallas_call(
        _y_body,
        grid=(N // NB,),
        in_specs=[
            pl.BlockSpec((NB, D), lambda i: (i, 0)),
            pl.BlockSpec((D, D), lambda i: (0, 0)),
            pl.BlockSpec((nc, NB, 16), lambda i: (0, i, 0)),
        ],
        out_specs=[
            pl.BlockSpec((NB, D), lambda i: (i, 0)),
            pl.BlockSpec((NB, 16), lambda i: (i, 0)),
        ],
        out_shape=[
            jax.ShapeDtypeStruct((N, D), F32),
            jax.ShapeDtypeStruct((N, 16), F32),
        ],
    )(x, W, degparts)


def _agg_call(nc, ns):
    nw = nc * ns
    rps = N // ns

    def body(y_hbm, a16_hbm, row_hbm, col_hbm, zer_hbm, out_hbm,
             ridx_v, cidx_v, a16_v, rows_v, h_sh, sem):
        cid = lax.axis_index("c")
        sid = lax.axis_index("s")
        wid = sid * nc + cid
        pltpu.sync_copy(zer_hbm, h_sh.at[pl.ds(sid * rps, rps)])
        plsc.subcore_barrier()

        @pl.loop(wid, NCH, step=nw)
        def _(c):
            base = c * CH
            pltpu.sync_copy(row_hbm.at[pl.ds(base, CH)], ridx_v)
            pltpu.sync_copy(col_hbm.at[pl.ds(base, CH)], cidx_v)
            pltpu.sync_copy(a16_hbm.at[pl.ds(base, CH)], a16_v)
            pltpu.async_copy(y_hbm.at[ridx_v], rows_v, sem).wait()
            for e in range(CH):
                a = a16_v[e, pl.ds(0, 16)]
                for k in range(D // 16):
                    sl = pl.ds(k * 16, 16)
                    rows_v[e, sl] = rows_v[e, sl] * a
            pltpu.sync_copy(rows_v, h_sh.at[cidx_v], add=True)

        plsc.subcore_barrier()
        pltpu.sync_copy(h_sh.at[pl.ds(sid * rps, rps)],
                        out_hbm.at[cid, pl.ds(sid * rps, rps)])

    mesh = plsc.VectorSubcoreMesh(core_axis_name="c", subcore_axis_name="s")
    return pl.kernel(
        body,
        out_type=jax.ShapeDtypeStruct((nc, N, D), F32),
        mesh=mesh,
        scratch_types=[
            pltpu.VMEM((CH,), jnp.int32),
            pltpu.VMEM((CH,), jnp.int32),
            pltpu.VMEM((CH, 16), F32),
            pltpu.VMEM((CH, D), F32),
            pltpu.VMEM_SHARED((N, D), F32),
            pltpu.SemaphoreType.DMA,
        ],
    )


def _out_body(hp_ref, dg_ref, x_ref, b_ref, g_ref, be_ref, o_ref):
    agg = jnp.sum(hp_ref[...], axis=0)
    deg = dg_ref[...]
    dis = jnp.where(deg > 0, lax.rsqrt(deg), 0.0)[:, 0:1]
    h = agg * dis + b_ref[...]
    mu = jnp.mean(h, axis=-1, keepdims=True)
    var = jnp.mean((h - mu) ** 2, axis=-1, keepdims=True)
    h = (h - mu) * lax.rsqrt(var + 1e-5) * g_ref[...] + be_ref[...]
    h = h * jax.nn.sigmoid(h)
    o_ref[...] = h + x_ref[...]


def _out_call(nc, hparts, degsum, x, b, gamma, beta):
    return pl.pallas_call(
        _out_body,
        grid=(N // NB,),
        in_specs=[
            pl.BlockSpec((nc, NB, D), lambda i: (0, i, 0)),
            pl.BlockSpec((NB, 16), lambda i: (i, 0)),
            pl.BlockSpec((NB, D), lambda i: (i, 0)),
            pl.BlockSpec((1, D), lambda i: (0, 0)),
            pl.BlockSpec((1, D), lambda i: (0, 0)),
            pl.BlockSpec((1, D), lambda i: (0, 0)),
        ],
        out_specs=pl.BlockSpec((NB, D), lambda i: (i, 0)),
        out_shape=jax.ShapeDtypeStruct((N, D), F32),
    )(hparts, degsum, x, b.reshape(1, D), gamma.reshape(1, D),
      beta.reshape(1, D))


def kernel(x, edge_index, edge_attr, W, b, W1, b1, W2, b2, gamma, beta):
    info = plsc.get_sparse_core_info()
    nc, ns = info.num_cores, info.num_subcores

    row = edge_index[0]
    col = edge_index[1]

    alpha16 = _alpha16(edge_attr, W1, b1, W2, b2)

    zeros16 = jnp.zeros((N // ns, 16), F32)
    zeros128 = jnp.zeros((N // ns, D), F32)

    degparts = _deg_call(nc, ns)(alpha16, col, zeros16)
    y, degsum = _y_call(nc, x, W, degparts)
    hparts = _agg_call(nc, ns)(y, alpha16, row, col, zeros128)
    return _out_call(nc, hparts, degsum, x, b, gamma, beta)


# speedup vs baseline: 4.5050x; 4.5050x over previous
import functools

import jax
import jax.numpy as jnp
from jax import lax
from jax.experimental import pallas as pl
from jax.experimental.pallas import tpu as pltpu
from jax.experimental.pallas import tpu_sc as plsc

N = 10000
NP = 10240
E = 320000
D = 128
D_EDGE = 16
HIDDEN = 32

CH = 128
NCH = E // CH
EB = 4000
NB = 1000

F32 = jnp.float32


def _alpha_body(ea_ref, w1_ref, b1_ref, w2t_ref, b2_ref, out_ref):
    ea = ea_ref[...]
    h1 = jnp.dot(ea, w1_ref[...], preferred_element_type=F32,
                 precision=lax.Precision.HIGHEST) + b1_ref[...]
    h1 = h1 * jax.nn.sigmoid(h1)
    z = jnp.sum(h1 * w2t_ref[...], axis=1, keepdims=True) + b2_ref[...]
    a = jax.nn.sigmoid(z)
    out_ref[...] = jnp.broadcast_to(a, (EB, 16))


def _alpha16(edge_attr, W1, b1, W2, b2):
    return pl.pallas_call(
        _alpha_body,
        grid=(E // EB,),
        in_specs=[
            pl.BlockSpec((EB, D_EDGE), lambda i: (i, 0)),
            pl.BlockSpec((D_EDGE, HIDDEN), lambda i: (0, 0)),
            pl.BlockSpec((1, HIDDEN), lambda i: (0, 0)),
            pl.BlockSpec((1, HIDDEN), lambda i: (0, 0)),
            pl.BlockSpec((1, 1), lambda i: (0, 0)),
        ],
        out_specs=pl.BlockSpec((EB, 16), lambda i: (i, 0)),
        out_shape=jax.ShapeDtypeStruct((E, 16), F32),
    )(edge_attr, W1, b1.reshape(1, HIDDEN), W2.reshape(1, HIDDEN),
      b2.reshape(1, 1))


def _deg_call(nc, ns):
    nw = nc * ns
    rps = NP // ns

    def body(a16_hbm, col_hbm, zer_hbm, out_hbm, col_v, a16_v, rows_v, deg_sh):
        cid = lax.axis_index("c")
        sid = lax.axis_index("s")
        wid = sid * nc + cid
        pltpu.sync_copy(zer_hbm, deg_sh.at[pl.ds(sid * rps, rps)])
        zv = jnp.zeros((16,), F32)
        for e in range(CH):
            for k in range(D // 16):
                rows_v[e, pl.ds(k * 16, 16)] = zv
        plsc.subcore_barrier()

        @pl.loop(wid, NCH, step=nw)
        def _(c):
            base = c * CH
            pltpu.sync_copy(col_hbm.at[pl.ds(base, CH)], col_v)
            pltpu.sync_copy(a16_hbm.at[pl.ds(base, CH)], a16_v)
            for e in range(CH):
                rows_v[e, pl.ds(0, 16)] = a16_v[e, pl.ds(0, 16)]
            pltpu.sync_copy(rows_v, deg_sh.at[col_v], add=True)

        plsc.subcore_barrier()
        pltpu.sync_copy(deg_sh.at[pl.ds(sid * rps, rps)],
                        out_hbm.at[cid, pl.ds(sid * rps, rps)])

    mesh = plsc.VectorSubcoreMesh(core_axis_name="c", subcore_axis_name="s")
    return pl.kernel(
        body,
        out_type=jax.ShapeDtypeStruct((nc, NP, D), F32),
        mesh=mesh,
        scratch_types=[
            pltpu.VMEM((CH,), jnp.int32),
            pltpu.VMEM((CH, 16), F32),
            pltpu.VMEM((CH, D), F32),
            pltpu.VMEM_SHARED((NP, D), F32),
        ],
    )


def _y_body(x_ref, w_ref, dp_ref, y_ref, dg_ref):
    xw = jnp.dot(x_ref[...], w_ref[...], preferred_element_type=F32,
                 precision=lax.Precision.HIGHEST)
    deg = jnp.sum(dp_ref[:, :, 0:16], axis=0)
    dis = jnp.where(deg > 0, lax.rsqrt(deg), 0.0)
    y_ref[...] = xw * dis[:, 0:1]
    dg_ref[...] = deg


def _y_call(nc, x, W, degparts):
    return pl.pallas_call(
        _y_body,
        grid=(N // NB,),
        in_specs=[
            pl.BlockSpec((NB, D), lambda i: (i, 0)),
            pl.BlockSpec((D, D), lambda i: (0, 0)),
            pl.BlockSpec((nc, NB, D), lambda i: (0, i, 0)),
        ],
        out_specs=[
            pl.BlockSpec((NB, D), lambda i: (i, 0)),
            pl.BlockSpec((NB, 16), lambda i: (i, 0)),
        ],
        out_shape=[
            jax.ShapeDtypeStruct((N, D), F32),
            jax.ShapeDtypeStruct((N, 16), F32),
        ],
    )(x, W, degparts)


def _agg_call(nc, ns):
    nw = nc * ns
    rps = NP // ns

    def body(y_hbm, a16_hbm, row_hbm, col_hbm, zer_hbm, out_hbm,
             ridx_v, cidx_v, a16_v, rows_v, h_sh, sem):
        cid = lax.axis_index("c")
        sid = lax.axis_index("s")
        wid = sid * nc + cid
        pltpu.sync_copy(zer_hbm, h_sh.at[pl.ds(sid * rps, rps)])
        plsc.subcore_barrier()

        @pl.loop(wid, NCH, step=nw)
        def _(c):
            base = c * CH
            pltpu.sync_copy(row_hbm.at[pl.ds(base, CH)], ridx_v)
            pltpu.sync_copy(col_hbm.at[pl.ds(base, CH)], cidx_v)
            pltpu.sync_copy(a16_hbm.at[pl.ds(base, CH)], a16_v)
            pltpu.async_copy(y_hbm.at[ridx_v], rows_v, sem).wait()
            for e in range(CH):
                a = a16_v[e, pl.ds(0, 16)]
                for k in range(D // 16):
                    sl = pl.ds(k * 16, 16)
                    rows_v[e, sl] = rows_v[e, sl] * a
            pltpu.sync_copy(rows_v, h_sh.at[cidx_v], add=True)

        plsc.subcore_barrier()
        pltpu.sync_copy(h_sh.at[pl.ds(sid * rps, rps)],
                        out_hbm.at[cid, pl.ds(sid * rps, rps)])

    mesh = plsc.VectorSubcoreMesh(core_axis_name="c", subcore_axis_name="s")
    return pl.kernel(
        body,
        out_type=jax.ShapeDtypeStruct((nc, NP, D), F32),
        mesh=mesh,
        scratch_types=[
            pltpu.VMEM((CH,), jnp.int32),
            pltpu.VMEM((CH,), jnp.int32),
            pltpu.VMEM((CH, 16), F32),
            pltpu.VMEM((CH, D), F32),
            pltpu.VMEM_SHARED((NP, D), F32),
            pltpu.SemaphoreType.DMA,
        ],
    )


def _out_body(hp_ref, dg_ref, x_ref, b_ref, g_ref, be_ref, o_ref):
    agg = jnp.sum(hp_ref[...], axis=0)
    deg = dg_ref[...]
    dis = jnp.where(deg > 0, lax.rsqrt(deg), 0.0)[:, 0:1]
    h = agg * dis + b_ref[...]
    mu = jnp.mean(h, axis=-1, keepdims=True)
    var = jnp.mean((h - mu) ** 2, axis=-1, keepdims=True)
    h = (h - mu) * lax.rsqrt(var + 1e-5) * g_ref[...] + be_ref[...]
    h = h * jax.nn.sigmoid(h)
    o_ref[...] = h + x_ref[...]


def _out_call(nc, hparts, degsum, x, b, gamma, beta):
    return pl.pallas_call(
        _out_body,
        grid=(N // NB,),
        in_specs=[
            pl.BlockSpec((nc, NB, D), lambda i: (0, i, 0)),
            pl.BlockSpec((NB, 16), lambda i: (i, 0)),
            pl.BlockSpec((NB, D), lambda i: (i, 0)),
            pl.BlockSpec((1, D), lambda i: (0, 0)),
            pl.BlockSpec((1, D), lambda i: (0, 0)),
            pl.BlockSpec((1, D), lambda i: (0, 0)),
        ],
        out_specs=pl.BlockSpec((NB, D), lambda i: (i, 0)),
        out_shape=jax.ShapeDtypeStruct((N, D), F32),
    )(hparts, degsum, x, b.reshape(1, D), gamma.reshape(1, D),
      beta.reshape(1, D))


def kernel(x, edge_index, edge_attr, W, b, W1, b1, W2, b2, gamma, beta):
    info = plsc.get_sparse_core_info()
    nc, ns = info.num_cores, info.num_subcores

    row = edge_index[0]
    col = edge_index[1]

    alpha16 = _alpha16(edge_attr, W1, b1, W2, b2)

    zeros128 = jnp.zeros((NP // ns, D), F32)

    degparts = _deg_call(nc, ns)(alpha16, col, zeros128)
    y, degsum = _y_call(nc, x, W, degparts)
    hparts = _agg_call(nc, ns)(y, alpha16, row, col, zeros128)
    return _out_call(nc, hparts, degsum, x, b, gamma, beta)


# speedup vs baseline: 7.3592x; 1.6336x over previous
import functools

import jax
import jax.numpy as jnp
from jax import lax
from jax.experimental import pallas as pl
from jax.experimental.pallas import tpu as pltpu
from jax.experimental.pallas import tpu_sc as plsc

N = 10000
NP = 10240
E = 320000
D = 128
D_EDGE = 16
HIDDEN = 32

CH = 128
NCH = E // CH
EB = 4000
NB = 1000

F32 = jnp.float32


E8 = E // 8
BB = 4000


def _alpha_body(ea_ref, w1_ref, b1_ref, w2_ref, s_ref, spl_ref, b2_ref,
                out_ref):
    hi = lax.Precision.HIGHEST
    h1 = jnp.dot(ea_ref[...], w1_ref[...], preferred_element_type=F32,
                 precision=hi) + b1_ref[...]
    h1 = h1 * jax.nn.sigmoid(h1)
    t = h1 * w2_ref[...]
    z8 = jnp.dot(t, s_ref[...], preferred_element_type=F32,
                 precision=hi) + b2_ref[...]
    a8 = jax.nn.sigmoid(z8)
    out_ref[...] = jnp.dot(a8, spl_ref[...], preferred_element_type=F32,
                           precision=hi)


def _alpha16(edge_attr, W1, b1, W2, b2):
    ea_p = edge_attr.reshape(E8, 8 * D_EDGE)
    eye8 = jnp.eye(8, dtype=F32)
    w1big = jnp.einsum("ij,kl->ikjl", eye8, W1).reshape(128, 8 * HIDDEN)
    b1big = jnp.tile(b1, 8).reshape(1, 8 * HIDDEN)
    w2big = jnp.tile(W2[:, 0], 8).reshape(1, 8 * HIDDEN)
    sel = jnp.einsum("ij,k->ikj", eye8, jnp.ones((HIDDEN,), F32))
    sel = sel.reshape(8 * HIDDEN, 8)
    spl = jnp.einsum("ij,k->ijk", eye8, jnp.ones((16,), F32))
    spl = spl.reshape(8, 128)
    b2big = jnp.broadcast_to(b2.reshape(1, 1), (1, 8))
    return pl.pallas_call(
        _alpha_body,
        grid=(E8 // BB,),
        in_specs=[
            pl.BlockSpec((BB, 128), lambda i: (i, 0)),
            pl.BlockSpec((128, 8 * HIDDEN), lambda i: (0, 0)),
            pl.BlockSpec((1, 8 * HIDDEN), lambda i: (0, 0)),
            pl.BlockSpec((1, 8 * HIDDEN), lambda i: (0, 0)),
            pl.BlockSpec((8 * HIDDEN, 8), lambda i: (0, 0)),
            pl.BlockSpec((8, 128), lambda i: (0, 0)),
            pl.BlockSpec((1, 8), lambda i: (0, 0)),
        ],
        out_specs=pl.BlockSpec((BB, 128), lambda i: (i, 0)),
        out_shape=jax.ShapeDtypeStruct((E8, 128), F32),
    )(ea_p, w1big, b1big, w2big, sel, spl, b2big)


def _deg_call(nc, ns):
    nw = nc * ns
    rps = NP // ns

    def body(a16_hbm, col_hbm, zer_hbm, out_hbm, col_v, a16_v, rows_v, deg_sh):
        cid = lax.axis_index("c")
        sid = lax.axis_index("s")
        wid = sid * nc + cid
        pltpu.sync_copy(zer_hbm, deg_sh.at[pl.ds(sid * rps, rps)])
        zv = jnp.zeros((16,), F32)
        for e in range(CH):
            for k in range(D // 16):
                rows_v[e, pl.ds(k * 16, 16)] = zv
        plsc.subcore_barrier()

        @pl.loop(wid, NCH, step=nw)
        def _(c):
            base = c * CH
            pltpu.sync_copy(col_hbm.at[pl.ds(base, CH)], col_v)
            pltpu.sync_copy(a16_hbm.at[pl.ds(c * (CH // 8), CH // 8)], a16_v)
            for e in range(CH):
                rows_v[e, pl.ds(0, 16)] = a16_v[e // 8, pl.ds((e % 8) * 16, 16)]
            pltpu.sync_copy(rows_v, deg_sh.at[col_v], add=True)

        plsc.subcore_barrier()
        pltpu.sync_copy(deg_sh.at[pl.ds(sid * rps, rps)],
                        out_hbm.at[cid, pl.ds(sid * rps, rps)])

    mesh = plsc.VectorSubcoreMesh(core_axis_name="c", subcore_axis_name="s")
    return pl.kernel(
        body,
        out_type=jax.ShapeDtypeStruct((nc, NP, D), F32),
        mesh=mesh,
        scratch_types=[
            pltpu.VMEM((CH,), jnp.int32),
            pltpu.VMEM((CH // 8, 128), F32),
            pltpu.VMEM((CH, D), F32),
            pltpu.VMEM_SHARED((NP, D), F32),
        ],
    )


def _y_body(x_ref, w_ref, dp_ref, y_ref, dg_ref):
    xw = jnp.dot(x_ref[...], w_ref[...], preferred_element_type=F32,
                 precision=lax.Precision.HIGHEST)
    deg = jnp.sum(dp_ref[:, :, 0:16], axis=0)
    dis = jnp.where(deg > 0, lax.rsqrt(deg), 0.0)
    y_ref[...] = xw * dis[:, 0:1]
    dg_ref[...] = deg


def _y_call(nc, x, W, degparts):
    return pl.pallas_call(
        _y_body,
        grid=(N // NB,),
        in_specs=[
            pl.BlockSpec((NB, D), lambda i: (i, 0)),
            pl.BlockSpec((D, D), lambda i: (0, 0)),
            pl.BlockSpec((nc, NB, D), lambda i: (0, i, 0)),
        ],
        out_specs=[
            pl.BlockSpec((NB, D), lambda i: (i, 0)),
            pl.BlockSpec((NB, 16), lambda i: (i, 0)),
        ],
        out_shape=[
            jax.ShapeDtypeStruct((N, D), F32),
            jax.ShapeDtypeStruct((N, 16), F32),
        ],
    )(x, W, degparts)


def _agg_call(nc, ns):
    nw = nc * ns
    rps = NP // ns

    def body(y_hbm, a16_hbm, row_hbm, col_hbm, zer_hbm, out_hbm,
             ridx_v, cidx_v, a16_v, rows_v, h_sh, sem):
        cid = lax.axis_index("c")
        sid = lax.axis_index("s")
        wid = sid * nc + cid
        pltpu.sync_copy(zer_hbm, h_sh.at[pl.ds(sid * rps, rps)])
        plsc.subcore_barrier()

        @pl.loop(wid, NCH, step=nw)
        def _(c):
            base = c * CH
            pltpu.sync_copy(row_hbm.at[pl.ds(base, CH)], ridx_v)
            pltpu.sync_copy(col_hbm.at[pl.ds(base, CH)], cidx_v)
            pltpu.sync_copy(a16_hbm.at[pl.ds(c * (CH // 8), CH // 8)], a16_v)
            pltpu.async_copy(y_hbm.at[ridx_v], rows_v, sem).wait()
            for e in range(CH):
                a = a16_v[e // 8, pl.ds((e % 8) * 16, 16)]
                for k in range(D // 16):
                    sl = pl.ds(k * 16, 16)
                    rows_v[e, sl] = rows_v[e, sl] * a
            pltpu.sync_copy(rows_v, h_sh.at[cidx_v], add=True)

        plsc.subcore_barrier()
        pltpu.sync_copy(h_sh.at[pl.ds(sid * rps, rps)],
                        out_hbm.at[cid, pl.ds(sid * rps, rps)])

    mesh = plsc.VectorSubcoreMesh(core_axis_name="c", subcore_axis_name="s")
    return pl.kernel(
        body,
        out_type=jax.ShapeDtypeStruct((nc, NP, D), F32),
        mesh=mesh,
        scratch_types=[
            pltpu.VMEM((CH,), jnp.int32),
            pltpu.VMEM((CH,), jnp.int32),
            pltpu.VMEM((CH // 8, 128), F32),
            pltpu.VMEM((CH, D), F32),
            pltpu.VMEM_SHARED((NP, D), F32),
            pltpu.SemaphoreType.DMA,
        ],
    )


def _out_body(hp_ref, dg_ref, x_ref, b_ref, g_ref, be_ref, o_ref):
    agg = jnp.sum(hp_ref[...], axis=0)
    deg = dg_ref[...]
    dis = jnp.where(deg > 0, lax.rsqrt(deg), 0.0)[:, 0:1]
    h = agg * dis + b_ref[...]
    mu = jnp.mean(h, axis=-1, keepdims=True)
    var = jnp.mean((h - mu) ** 2, axis=-1, keepdims=True)
    h = (h - mu) * lax.rsqrt(var + 1e-5) * g_ref[...] + be_ref[...]
    h = h * jax.nn.sigmoid(h)
    o_ref[...] = h + x_ref[...]


def _out_call(nc, hparts, degsum, x, b, gamma, beta):
    return pl.pallas_call(
        _out_body,
        grid=(N // NB,),
        in_specs=[
            pl.BlockSpec((nc, NB, D), lambda i: (0, i, 0)),
            pl.BlockSpec((NB, 16), lambda i: (i, 0)),
            pl.BlockSpec((NB, D), lambda i: (i, 0)),
            pl.BlockSpec((1, D), lambda i: (0, 0)),
            pl.BlockSpec((1, D), lambda i: (0, 0)),
            pl.BlockSpec((1, D), lambda i: (0, 0)),
        ],
        out_specs=pl.BlockSpec((NB, D), lambda i: (i, 0)),
        out_shape=jax.ShapeDtypeStruct((N, D), F32),
    )(hparts, degsum, x, b.reshape(1, D), gamma.reshape(1, D),
      beta.reshape(1, D))


def kernel(x, edge_index, edge_attr, W, b, W1, b1, W2, b2, gamma, beta):
    info = plsc.get_sparse_core_info()
    nc, ns = info.num_cores, info.num_subcores

    row = edge_index[0]
    col = edge_index[1]

    alpha16 = _alpha16(edge_attr, W1, b1, W2, b2)

    zeros128 = jnp.zeros((NP // ns, D), F32)

    degparts = _deg_call(nc, ns)(alpha16, col, zeros128)
    y, degsum = _y_call(nc, x, W, degparts)
    hparts = _agg_call(nc, ns)(y, alpha16, row, col, zeros128)
    return _out_call(nc, hparts, degsum, x, b, gamma, beta)


# speedup vs baseline: 9.7361x; 1.3230x over previous
import functools

import jax
import jax.numpy as jnp
from jax import lax
from jax.experimental import pallas as pl
from jax.experimental.pallas import tpu as pltpu
from jax.experimental.pallas import tpu_sc as plsc

N = 10000
NP = 10240
E = 320000
D = 128
D_EDGE = 16
HIDDEN = 32

CH = 128
G = CH // 128
NCH = E // CH
NB = 1000

F32 = jnp.float32


E8 = E // 8
BB = 4000


def _alpha_body(ea_ref, w1_ref, b1_ref, w2_ref, s_ref, spl_ref, b2_ref,
                out_ref):
    hi = lax.Precision.HIGHEST
    h1 = jnp.dot(ea_ref[...], w1_ref[...], preferred_element_type=F32,
                 precision=hi) + b1_ref[...]
    h1 = h1 * jax.nn.sigmoid(h1)
    t = h1 * w2_ref[...]
    z8 = jnp.dot(t, s_ref[...], preferred_element_type=F32,
                 precision=hi) + b2_ref[...]
    a8 = jax.nn.sigmoid(z8)
    out_ref[...] = jnp.dot(a8, spl_ref[...], preferred_element_type=F32,
                           precision=hi)


def _alpha16(edge_attr, W1, b1, W2, b2):
    ea_p = edge_attr.reshape(E8, 8 * D_EDGE)
    eye8 = jnp.eye(8, dtype=F32)
    w1big = jnp.einsum("ij,kl->ikjl", eye8, W1).reshape(128, 8 * HIDDEN)
    b1big = jnp.tile(b1, 8).reshape(1, 8 * HIDDEN)
    w2big = jnp.tile(W2[:, 0], 8).reshape(1, 8 * HIDDEN)
    sel = jnp.einsum("ij,k->ikj", eye8, jnp.ones((HIDDEN,), F32))
    sel = sel.reshape(8 * HIDDEN, 8)
    spl = jnp.einsum("ij,k->ijk", eye8, jnp.ones((16,), F32))
    spl = spl.reshape(8, 128)
    b2big = jnp.broadcast_to(b2.reshape(1, 1), (1, 8))
    return pl.pallas_call(
        _alpha_body,
        grid=(E8 // BB,),
        in_specs=[
            pl.BlockSpec((BB, 128), lambda i: (i, 0)),
            pl.BlockSpec((128, 8 * HIDDEN), lambda i: (0, 0)),
            pl.BlockSpec((1, 8 * HIDDEN), lambda i: (0, 0)),
            pl.BlockSpec((1, 8 * HIDDEN), lambda i: (0, 0)),
            pl.BlockSpec((8 * HIDDEN, 8), lambda i: (0, 0)),
            pl.BlockSpec((8, 128), lambda i: (0, 0)),
            pl.BlockSpec((1, 8), lambda i: (0, 0)),
        ],
        out_specs=pl.BlockSpec((BB, 128), lambda i: (i, 0)),
        out_shape=jax.ShapeDtypeStruct((E8, 128), F32),
    )(ea_p, w1big, b1big, w2big, sel, spl, b2big)


def _issue_loads(rc_hbm, a16_hbm, c, rc_v, a_v, sem):
    pltpu.async_copy(rc_hbm.at[pl.ds(c * 8, 8)], rc_v, sem)
    pltpu.async_copy(a16_hbm.at[pl.ds(c * (CH // 8), CH // 8)], a_v, sem)


def _wait_loads(rc_hbm, a16_hbm, rc_v, a_v, sem):
    pltpu.make_async_copy(rc_hbm.at[pl.ds(0, 8)], rc_v, sem).wait()
    pltpu.make_async_copy(a16_hbm.at[pl.ds(0, CH // 8)], a_v, sem).wait()


def _issue_scatter(tbl_sh, rc_v, rows_v, sem):
    for g in range(G):
        pltpu.async_copy(rows_v.at[pl.ds(g * 128, 128)],
                         tbl_sh.at[rc_v.at[G + g]], sem, add=True)


def _wait_scatter(tbl_sh, rc_v, rows_v, sem):
    for g in range(G):
        pltpu.make_async_copy(rows_v.at[pl.ds(g * 128, 128)],
                              tbl_sh.at[rc_v.at[G + g]], sem).wait()


def _issue_gather(y_hbm, rc_v, rows_v, sem):
    for g in range(G):
        pltpu.async_copy(y_hbm.at[rc_v.at[g]],
                         rows_v.at[pl.ds(g * 128, 128)], sem)


def _wait_gather(y_hbm, rc_v, rows_v, sem):
    for g in range(G):
        pltpu.make_async_copy(y_hbm.at[rc_v.at[g]],
                              rows_v.at[pl.ds(g * 128, 128)], sem).wait()


def _deg_call(nc, ns):
    nw = nc * ns
    rps = NP // ns

    def fill(rows_v, a_v):
        @pl.loop(0, CH, step=8)
        def _(e0):
            for j in range(8):
                rows_v[e0 + j, pl.ds(0, 16)] = a_v[e0 // 8, pl.ds(j * 16, 16)]

    def body(a16_hbm, rc_hbm, zer_hbm, out_hbm,
             rcA, aA, rowsA, rcB, aB, rowsB, deg_sh, slA, slB, ssA, ssB):
        cid = lax.axis_index("c")
        sid = lax.axis_index("s")
        wid = sid * nc + cid
        nk = (NCH - wid + nw - 1) // nw
        pltpu.sync_copy(zer_hbm, deg_sh.at[pl.ds(sid * rps, rps)])
        zv = jnp.zeros((16,), F32)

        @pl.loop(0, CH)
        def _(e):
            for k in range(D // 16):
                rowsA[e, pl.ds(k * 16, 16)] = zv
                rowsB[e, pl.ds(k * 16, 16)] = zv

        plsc.subcore_barrier()
        pltpu.sync_copy(rc_hbm.at[pl.ds(wid * 8, 8)], rcA)
        pltpu.sync_copy(a16_hbm.at[pl.ds(wid * (CH // 8), CH // 8)], aA)

        @pl.loop(wid, NCH - nw, step=2 * nw)
        def _(c):
            @pl.when(c > wid)
            def _():
                _wait_scatter(deg_sh, rcB, rowsB, ssB)

            _issue_loads(rc_hbm, a16_hbm, c + nw, rcB, aB, slB)
            fill(rowsA, aA)
            _issue_scatter(deg_sh, rcA, rowsA, ssA)
            _wait_loads(rc_hbm, a16_hbm, rcB, aB, slB)
            fill(rowsB, aB)
            _issue_scatter(deg_sh, rcB, rowsB, ssB)
            _wait_scatter(deg_sh, rcA, rowsA, ssA)
            cn = lax.select(c + 2 * nw < NCH, c + 2 * nw, c)
            _issue_loads(rc_hbm, a16_hbm, cn, rcA, aA, slA)
            _wait_loads(rc_hbm, a16_hbm, rcA, aA, slA)

        _wait_scatter(deg_sh, rcB, rowsB, ssB)

        @pl.when(nk % 2 == 1)
        def _():
            fill(rowsA, aA)
            _issue_scatter(deg_sh, rcA, rowsA, ssA)
            _wait_scatter(deg_sh, rcA, rowsA, ssA)

        plsc.subcore_barrier()
        pltpu.sync_copy(deg_sh.at[pl.ds(sid * rps, rps)],
                        out_hbm.at[cid, pl.ds(sid * rps, rps)])

    mesh = plsc.VectorSubcoreMesh(core_axis_name="c", subcore_axis_name="s")
    return pl.kernel(
        body,
        out_type=jax.ShapeDtypeStruct((nc, NP, D), F32),
        mesh=mesh,
        scratch_types=[
            pltpu.VMEM((8, 128), jnp.int32),
            pltpu.VMEM((CH // 8, 128), F32),
            pltpu.VMEM((CH, D), F32),
            pltpu.VMEM((8, 128), jnp.int32),
            pltpu.VMEM((CH // 8, 128), F32),
            pltpu.VMEM((CH, D), F32),
            pltpu.VMEM_SHARED((NP, D), F32),
            pltpu.SemaphoreType.DMA,
            pltpu.SemaphoreType.DMA,
            pltpu.SemaphoreType.DMA,
            pltpu.SemaphoreType.DMA,
        ],
    )


def _y_body(x_ref, w_ref, dp_ref, y_ref, dg_ref):
    xw = jnp.dot(x_ref[...], w_ref[...], preferred_element_type=F32,
                 precision=lax.Precision.HIGHEST)
    deg = jnp.sum(dp_ref[:, :, 0:16], axis=0)
    dis = jnp.where(deg > 0, lax.rsqrt(deg), 0.0)
    y_ref[...] = xw * dis[:, 0:1]
    dg_ref[...] = deg


def _y_call(nc, x, W, degparts):
    return pl.pallas_call(
        _y_body,
        grid=(N // NB,),
        in_specs=[
            pl.BlockSpec((NB, D), lambda i: (i, 0)),
            pl.BlockSpec((D, D), lambda i: (0, 0)),
            pl.BlockSpec((nc, NB, D), lambda i: (0, i, 0)),
        ],
        out_specs=[
            pl.BlockSpec((NB, D), lambda i: (i, 0)),
            pl.BlockSpec((NB, 16), lambda i: (i, 0)),
        ],
        out_shape=[
            jax.ShapeDtypeStruct((N, D), F32),
            jax.ShapeDtypeStruct((N, 16), F32),
        ],
    )(x, W, degparts)


def _agg_call(nc, ns):
    nw = nc * ns
    rps = NP // ns

    def scale(rows_v, a_v):
        @pl.loop(0, CH, step=8)
        def _(e0):
            for j in range(8):
                a = a_v[e0 // 8, pl.ds(j * 16, 16)]
                for k in range(D // 16):
                    sl = pl.ds(k * 16, 16)
                    rows_v[e0 + j, sl] = rows_v[e0 + j, sl] * a

    def body(y_hbm, a16_hbm, rc_hbm, zer_hbm, out_hbm,
             rcA, aA, rowsA, rcB, aB, rowsB, h_sh,
             slA, slB, sgA, sgB, ssA, ssB):
        cid = lax.axis_index("c")
        sid = lax.axis_index("s")
        wid = sid * nc + cid
        nk = (NCH - wid + nw - 1) // nw
        pltpu.sync_copy(zer_hbm, h_sh.at[pl.ds(sid * rps, rps)])
        plsc.subcore_barrier()
        pltpu.sync_copy(rc_hbm.at[pl.ds(wid * 8, 8)], rcA)
        pltpu.sync_copy(a16_hbm.at[pl.ds(wid * (CH // 8), CH // 8)], aA)
        _issue_gather(y_hbm, rcA, rowsA, sgA)

        @pl.loop(wid, NCH - nw, step=2 * nw)
        def _(c):
            @pl.when(c > wid)
            def _():
                _wait_scatter(h_sh, rcB, rowsB, ssB)

            _issue_loads(rc_hbm, a16_hbm, c + nw, rcB, aB, slB)
            _wait_gather(y_hbm, rcA, rowsA, sgA)
            scale(rowsA, aA)
            _issue_scatter(h_sh, rcA, rowsA, ssA)
            _wait_loads(rc_hbm, a16_hbm, rcB, aB, slB)
            _issue_gather(y_hbm, rcB, rowsB, sgB)
            _wait_scatter(h_sh, rcA, rowsA, ssA)
            cn = lax.select(c + 2 * nw < NCH, c + 2 * nw, c)
            _issue_loads(rc_hbm, a16_hbm, cn, rcA, aA, slA)
            _wait_gather(y_hbm, rcB, rowsB, sgB)
            scale(rowsB, aB)
            _issue_scatter(h_sh, rcB, rowsB, ssB)
            _wait_loads(rc_hbm, a16_hbm, rcA, aA, slA)
            _issue_gather(y_hbm, rcA, rowsA, sgA)

        _wait_gather(y_hbm, rcA, rowsA, sgA)
        _wait_scatter(h_sh, rcB, rowsB, ssB)

        @pl.when(nk % 2 == 1)
        def _():
            scale(rowsA, aA)
            _issue_scatter(h_sh, rcA, rowsA, ssA)
            _wait_scatter(h_sh, rcA, rowsA, ssA)

        plsc.subcore_barrier()
        pltpu.sync_copy(h_sh.at[pl.ds(sid * rps, rps)],
                        out_hbm.at[cid, pl.ds(sid * rps, rps)])

    mesh = plsc.VectorSubcoreMesh(core_axis_name="c", subcore_axis_name="s")
    return pl.kernel(
        body,
        out_type=jax.ShapeDtypeStruct((nc, NP, D), F32),
        mesh=mesh,
        scratch_types=[
            pltpu.VMEM((8, 128), jnp.int32),
            pltpu.VMEM((CH // 8, 128), F32),
            pltpu.VMEM((CH, D), F32),
            pltpu.VMEM((8, 128), jnp.int32),
            pltpu.VMEM((CH // 8, 128), F32),
            pltpu.VMEM((CH, D), F32),
            pltpu.VMEM_SHARED((NP, D), F32),
            pltpu.SemaphoreType.DMA,
            pltpu.SemaphoreType.DMA,
            pltpu.SemaphoreType.DMA,
            pltpu.SemaphoreType.DMA,
            pltpu.SemaphoreType.DMA,
            pltpu.SemaphoreType.DMA,
        ],
    )


def _out_body(hp_ref, dg_ref, x_ref, b_ref, g_ref, be_ref, o_ref):
    agg = jnp.sum(hp_ref[...], axis=0)
    deg = dg_ref[...]
    dis = jnp.where(deg > 0, lax.rsqrt(deg), 0.0)[:, 0:1]
    h = agg * dis + b_ref[...]
    mu = jnp.mean(h, axis=-1, keepdims=True)
    var = jnp.mean((h - mu) ** 2, axis=-1, keepdims=True)
    h = (h - mu) * lax.rsqrt(var + 1e-5) * g_ref[...] + be_ref[...]
    h = h * jax.nn.sigmoid(h)
    o_ref[...] = h + x_ref[...]


def _out_call(nc, hparts, degsum, x, b, gamma, beta):
    return pl.pallas_call(
        _out_body,
        grid=(N // NB,),
        in_specs=[
            pl.BlockSpec((nc, NB, D), lambda i: (0, i, 0)),
            pl.BlockSpec((NB, 16), lambda i: (i, 0)),
            pl.BlockSpec((NB, D), lambda i: (i, 0)),
            pl.BlockSpec((1, D), lambda i: (0, 0)),
            pl.BlockSpec((1, D), lambda i: (0, 0)),
            pl.BlockSpec((1, D), lambda i: (0, 0)),
        ],
        out_specs=pl.BlockSpec((NB, D), lambda i: (i, 0)),
        out_shape=jax.ShapeDtypeStruct((N, D), F32),
    )(hparts, degsum, x, b.reshape(1, D), gamma.reshape(1, D),
      beta.reshape(1, D))


def kernel(x, edge_index, edge_attr, W, b, W1, b1, W2, b2, gamma, beta):
    info = plsc.get_sparse_core_info()
    nc, ns = info.num_cores, info.num_subcores

    row = edge_index[0]
    col = edge_index[1]

    alpha16 = _alpha16(edge_attr, W1, b1, W2, b2)

    zeros128 = jnp.zeros((NP // ns, D), F32)

    rc = jnp.concatenate(
        [row.reshape(NCH, G, 128), col.reshape(NCH, G, 128),
         jnp.zeros((NCH, 8 - 2 * G, 128), jnp.int32)],
        axis=1).reshape(NCH * 8, 128)

    degparts = _deg_call(nc, ns)(alpha16, rc, zeros128)
    y, degsum = _y_call(nc, x, W, degparts)
    hparts = _agg_call(nc, ns)(y, alpha16, rc, zeros128)
    return _out_call(nc, hparts, degsum, x, b, gamma, beta)


# speedup vs baseline: 9.8525x; 1.0120x over previous
import functools

import jax
import jax.numpy as jnp
from jax import lax
from jax.experimental import pallas as pl
from jax.experimental.pallas import tpu as pltpu
from jax.experimental.pallas import tpu_sc as plsc

N = 10000
NP = 10240
E = 320000
D = 128
D_EDGE = 16
HIDDEN = 32

CH = 128
G = CH // 128
NCH = E // CH
NB = 1000

F32 = jnp.float32


E8 = E // 8
BB = 4000


def _alpha_body(ea_ref, w1_ref, b1_ref, w2_ref, s_ref, spl_ref, b2_ref,
                out_ref):
    hi = lax.Precision.HIGHEST
    h1 = jnp.dot(ea_ref[...], w1_ref[...], preferred_element_type=F32,
                 precision=hi) + b1_ref[...]
    h1 = h1 * jax.nn.sigmoid(h1)
    t = h1 * w2_ref[...]
    z8 = jnp.dot(t, s_ref[...], preferred_element_type=F32,
                 precision=hi) + b2_ref[...]
    a8 = jax.nn.sigmoid(z8)
    out_ref[...] = jnp.dot(a8, spl_ref[...], preferred_element_type=F32,
                           precision=hi)


def _alpha16(edge_attr, W1, b1, W2, b2):
    ea_p = edge_attr.reshape(E8, 8 * D_EDGE)
    eye8 = jnp.eye(8, dtype=F32)
    w1big = jnp.einsum("ij,kl->ikjl", eye8, W1).reshape(128, 8 * HIDDEN)
    b1big = jnp.tile(b1, 8).reshape(1, 8 * HIDDEN)
    w2big = jnp.tile(W2[:, 0], 8).reshape(1, 8 * HIDDEN)
    sel = jnp.einsum("ij,k->ikj", eye8, jnp.ones((HIDDEN,), F32))
    sel = sel.reshape(8 * HIDDEN, 8)
    spl = jnp.einsum("ij,k->ijk", eye8, jnp.ones((16,), F32))
    spl = spl.reshape(8, 128)
    b2big = jnp.broadcast_to(b2.reshape(1, 1), (1, 8))
    return pl.pallas_call(
        _alpha_body,
        grid=(E8 // BB,),
        in_specs=[
            pl.BlockSpec((BB, 128), lambda i: (i, 0)),
            pl.BlockSpec((128, 8 * HIDDEN), lambda i: (0, 0)),
            pl.BlockSpec((1, 8 * HIDDEN), lambda i: (0, 0)),
            pl.BlockSpec((1, 8 * HIDDEN), lambda i: (0, 0)),
            pl.BlockSpec((8 * HIDDEN, 8), lambda i: (0, 0)),
            pl.BlockSpec((8, 128), lambda i: (0, 0)),
            pl.BlockSpec((1, 8), lambda i: (0, 0)),
        ],
        out_specs=pl.BlockSpec((BB, 128), lambda i: (i, 0)),
        out_shape=jax.ShapeDtypeStruct((E8, 128), F32),
        compiler_params=pltpu.CompilerParams(
            dimension_semantics=("parallel",)),
    )(ea_p, w1big, b1big, w2big, sel, spl, b2big)


def _issue_scatter(tbl_sh, c_v, rows_v, sem):
    pltpu.async_copy(rows_v, tbl_sh.at[c_v], sem, add=True)


def _wait_scatter(tbl_sh, c_v, rows_v, sem):
    pltpu.make_async_copy(rows_v, tbl_sh.at[c_v], sem).wait()


def _issue_gather(y_hbm, r_v, rows_v, sem):
    pltpu.async_copy(y_hbm.at[r_v], rows_v, sem)


def _wait_gather(y_hbm, r_v, rows_v, sem):
    pltpu.make_async_copy(y_hbm.at[r_v], rows_v, sem).wait()


def _deg_call(nc, ns):
    nw = nc * ns
    rps = NP // ns

    def fill(rows_v, a_v):
        @pl.loop(0, CH, step=8)
        def _(e0):
            for j in range(8):
                rows_v[e0 + j, pl.ds(0, 16)] = a_v[e0 // 8, pl.ds(j * 16, 16)]

    def load(c, c_v, a_v, sem):
        pltpu.async_copy(col_hbm_ref[0].at[pl.ds(c * CH, CH)], c_v, sem)
        pltpu.async_copy(a16_hbm_ref[0].at[pl.ds(c * (CH // 8), CH // 8)],
                         a_v, sem)

    def wait_load(c_v, a_v, sem):
        pltpu.make_async_copy(col_hbm_ref[0].at[pl.ds(0, CH)], c_v, sem).wait()
        pltpu.make_async_copy(a16_hbm_ref[0].at[pl.ds(0, CH // 8)], a_v,
                              sem).wait()

    col_hbm_ref = [None]
    a16_hbm_ref = [None]

    def body(a16_hbm, col_hbm, zer_hbm, out_hbm,
             cA, aA, rowsA, cB, aB, rowsB, deg_sh, slA, slB, ssA, ssB):
        col_hbm_ref[0] = col_hbm
        a16_hbm_ref[0] = a16_hbm
        cid = lax.axis_index("c")
        sid = lax.axis_index("s")
        wid = sid * nc + cid
        nk = (NCH - wid + nw - 1) // nw
        pltpu.sync_copy(zer_hbm, deg_sh.at[pl.ds(sid * rps, rps)])
        zv = jnp.zeros((16,), F32)

        @pl.loop(0, CH)
        def _(e):
            for k in range(D // 16):
                rowsA[e, pl.ds(k * 16, 16)] = zv
                rowsB[e, pl.ds(k * 16, 16)] = zv

        plsc.subcore_barrier()
        pltpu.sync_copy(col_hbm.at[pl.ds(wid * CH, CH)], cA)
        pltpu.sync_copy(a16_hbm.at[pl.ds(wid * (CH // 8), CH // 8)], aA)

        @pl.loop(wid, NCH - nw, step=2 * nw)
        def _(c):
            @pl.when(c > wid)
            def _():
                _wait_scatter(deg_sh, cB, rowsB, ssB)

            load(c + nw, cB, aB, slB)
            fill(rowsA, aA)
            _issue_scatter(deg_sh, cA, rowsA, ssA)
            wait_load(cB, aB, slB)
            fill(rowsB, aB)
            _issue_scatter(deg_sh, cB, rowsB, ssB)
            _wait_scatter(deg_sh, cA, rowsA, ssA)
            cn = lax.select(c + 2 * nw < NCH, c + 2 * nw, c)
            load(cn, cA, aA, slA)
            wait_load(cA, aA, slA)

        _wait_scatter(deg_sh, cB, rowsB, ssB)

        @pl.when(nk % 2 == 1)
        def _():
            fill(rowsA, aA)
            _issue_scatter(deg_sh, cA, rowsA, ssA)
            _wait_scatter(deg_sh, cA, rowsA, ssA)

        plsc.subcore_barrier()
        pltpu.sync_copy(deg_sh.at[pl.ds(sid * rps, rps)],
                        out_hbm.at[cid, pl.ds(sid * rps, rps)])

    mesh = plsc.VectorSubcoreMesh(core_axis_name="c", subcore_axis_name="s")
    return pl.kernel(
        body,
        out_type=jax.ShapeDtypeStruct((nc, NP, D), F32),
        mesh=mesh,
        scratch_types=[
            pltpu.VMEM((CH,), jnp.int32),
            pltpu.VMEM((CH // 8, 128), F32),
            pltpu.VMEM((CH, D), F32),
            pltpu.VMEM((CH,), jnp.int32),
            pltpu.VMEM((CH // 8, 128), F32),
            pltpu.VMEM((CH, D), F32),
            pltpu.VMEM_SHARED((NP, D), F32),
            pltpu.SemaphoreType.DMA,
            pltpu.SemaphoreType.DMA,
            pltpu.SemaphoreType.DMA,
            pltpu.SemaphoreType.DMA,
        ],
    )


def _y_body(x_ref, w_ref, dp_ref, y_ref, dg_ref):
    xw = jnp.dot(x_ref[...], w_ref[...], preferred_element_type=F32,
                 precision=lax.Precision.HIGHEST)
    deg = jnp.sum(dp_ref[:, :, 0:16], axis=0)
    dis = jnp.where(deg > 0, lax.rsqrt(deg), 0.0)
    y_ref[...] = xw * dis[:, 0:1]
    dg_ref[...] = deg


def _y_call(nc, x, W, degparts):
    return pl.pallas_call(
        _y_body,
        grid=(N // NB,),
        in_specs=[
            pl.BlockSpec((NB, D), lambda i: (i, 0)),
            pl.BlockSpec((D, D), lambda i: (0, 0)),
            pl.BlockSpec((nc, NB, D), lambda i: (0, i, 0)),
        ],
        out_specs=[
            pl.BlockSpec((NB, D), lambda i: (i, 0)),
            pl.BlockSpec((NB, 16), lambda i: (i, 0)),
        ],
        out_shape=[
            jax.ShapeDtypeStruct((N, D), F32),
            jax.ShapeDtypeStruct((N, 16), F32),
        ],
        compiler_params=pltpu.CompilerParams(
            dimension_semantics=("parallel",)),
    )(x, W, degparts)


def _agg_call(nc, ns):
    nw = nc * ns
    rps = NP // ns

    def scale(rows_v, a_v):
        @pl.loop(0, CH, step=8)
        def _(e0):
            for j in range(8):
                a = a_v[e0 // 8, pl.ds(j * 16, 16)]
                for k in range(D // 16):
                    sl = pl.ds(k * 16, 16)
                    rows_v[e0 + j, sl] = rows_v[e0 + j, sl] * a

    refs = {}

    def load(c, r_v, c_v, a_v, sem):
        pltpu.async_copy(refs["row"].at[pl.ds(c * CH, CH)], r_v, sem)
        pltpu.async_copy(refs["col"].at[pl.ds(c * CH, CH)], c_v, sem)
        pltpu.async_copy(refs["a16"].at[pl.ds(c * (CH // 8), CH // 8)],
                         a_v, sem)

    def wait_load(r_v, c_v, a_v, sem):
        pltpu.make_async_copy(refs["row"].at[pl.ds(0, CH)], r_v, sem).wait()
        pltpu.make_async_copy(refs["col"].at[pl.ds(0, CH)], c_v, sem).wait()
        pltpu.make_async_copy(refs["a16"].at[pl.ds(0, CH // 8)], a_v,
                              sem).wait()

    def body(y_hbm, a16_hbm, row_hbm, col_hbm, zer_hbm, out_hbm,
             rA, cA, aA, rowsA, rB, cB, aB, rowsB, h_sh,
             slA, slB, sgA, sgB, ssA, ssB):
        refs["row"] = row_hbm
        refs["col"] = col_hbm
        refs["a16"] = a16_hbm
        cid = lax.axis_index("c")
        sid = lax.axis_index("s")
        wid = sid * nc + cid
        nk = (NCH - wid + nw - 1) // nw
        pltpu.sync_copy(zer_hbm, h_sh.at[pl.ds(sid * rps, rps)])
        plsc.subcore_barrier()
        pltpu.sync_copy(row_hbm.at[pl.ds(wid * CH, CH)], rA)
        pltpu.sync_copy(col_hbm.at[pl.ds(wid * CH, CH)], cA)
        pltpu.sync_copy(a16_hbm.at[pl.ds(wid * (CH // 8), CH // 8)], aA)
        _issue_gather(y_hbm, rA, rowsA, sgA)

        @pl.loop(wid, NCH - nw, step=2 * nw)
        def _(c):
            @pl.when(c > wid)
            def _():
                _wait_scatter(h_sh, cB, rowsB, ssB)

            load(c + nw, rB, cB, aB, slB)
            _wait_gather(y_hbm, rA, rowsA, sgA)
            scale(rowsA, aA)
            _issue_scatter(h_sh, cA, rowsA, ssA)
            wait_load(rB, cB, aB, slB)
            _issue_gather(y_hbm, rB, rowsB, sgB)
            _wait_scatter(h_sh, cA, rowsA, ssA)
            cn = lax.select(c + 2 * nw < NCH, c + 2 * nw, c)
            load(cn, rA, cA, aA, slA)
            _wait_gather(y_hbm, rB, rowsB, sgB)
            scale(rowsB, aB)
            _issue_scatter(h_sh, cB, rowsB, ssB)
            wait_load(rA, cA, aA, slA)
            _issue_gather(y_hbm, rA, rowsA, sgA)

        _wait_gather(y_hbm, rA, rowsA, sgA)
        _wait_scatter(h_sh, cB, rowsB, ssB)

        @pl.when(nk % 2 == 1)
        def _():
            scale(rowsA, aA)
            _issue_scatter(h_sh, cA, rowsA, ssA)
            _wait_scatter(h_sh, cA, rowsA, ssA)

        plsc.subcore_barrier()
        pltpu.sync_copy(h_sh.at[pl.ds(sid * rps, rps)],
                        out_hbm.at[cid, pl.ds(sid * rps, rps)])

    mesh = plsc.VectorSubcoreMesh(core_axis_name="c", subcore_axis_name="s")
    return pl.kernel(
        body,
        out_type=jax.ShapeDtypeStruct((nc, NP, D), F32),
        mesh=mesh,
        scratch_types=[
            pltpu.VMEM((CH,), jnp.int32),
            pltpu.VMEM((CH,), jnp.int32),
            pltpu.VMEM((CH // 8, 128), F32),
            pltpu.VMEM((CH, D), F32),
            pltpu.VMEM((CH,), jnp.int32),
            pltpu.VMEM((CH,), jnp.int32),
            pltpu.VMEM((CH // 8, 128), F32),
            pltpu.VMEM((CH, D), F32),
            pltpu.VMEM_SHARED((NP, D), F32),
            pltpu.SemaphoreType.DMA,
            pltpu.SemaphoreType.DMA,
            pltpu.SemaphoreType.DMA,
            pltpu.SemaphoreType.DMA,
            pltpu.SemaphoreType.DMA,
            pltpu.SemaphoreType.DMA,
        ],
    )


def _out_body(hp_ref, dg_ref, x_ref, b_ref, g_ref, be_ref, o_ref):
    agg = jnp.sum(hp_ref[...], axis=0)
    deg = dg_ref[...]
    dis = jnp.where(deg > 0, lax.rsqrt(deg), 0.0)[:, 0:1]
    h = agg * dis + b_ref[...]
    mu = jnp.mean(h, axis=-1, keepdims=True)
    var = jnp.mean((h - mu) ** 2, axis=-1, keepdims=True)
    h = (h - mu) * lax.rsqrt(var + 1e-5) * g_ref[...] + be_ref[...]
    h = h * jax.nn.sigmoid(h)
    o_ref[...] = h + x_ref[...]


def _out_call(nc, hparts, degsum, x, b, gamma, beta):
    return pl.pallas_call(
        _out_body,
        grid=(N // NB,),
        in_specs=[
            pl.BlockSpec((nc, NB, D), lambda i: (0, i, 0)),
            pl.BlockSpec((NB, 16), lambda i: (i, 0)),
            pl.BlockSpec((NB, D), lambda i: (i, 0)),
            pl.BlockSpec((1, D), lambda i: (0, 0)),
            pl.BlockSpec((1, D), lambda i: (0, 0)),
            pl.BlockSpec((1, D), lambda i: (0, 0)),
        ],
        out_specs=pl.BlockSpec((NB, D), lambda i: (i, 0)),
        out_shape=jax.ShapeDtypeStruct((N, D), F32),
        compiler_params=pltpu.CompilerParams(
            dimension_semantics=("parallel",)),
    )(hparts, degsum, x, b.reshape(1, D), gamma.reshape(1, D),
      beta.reshape(1, D))


def kernel(x, edge_index, edge_attr, W, b, W1, b1, W2, b2, gamma, beta):
    info = plsc.get_sparse_core_info()
    nc, ns = info.num_cores, info.num_subcores

    row = edge_index[0]
    col = edge_index[1]

    alpha16 = _alpha16(edge_attr, W1, b1, W2, b2)

    zeros128 = jnp.zeros((NP // ns, D), F32)

    degparts = _deg_call(nc, ns)(alpha16, col, zeros128)
    y, degsum = _y_call(nc, x, W, degparts)
    hparts = _agg_call(nc, ns)(y, alpha16, row, col, zeros128)
    return _out_call(nc, hparts, degsum, x, b, gamma, beta)


# speedup vs baseline: 10.7273x; 1.0888x over previous
import dataclasses
import functools

import jax
import jax.numpy as jnp
from jax import lax
from jax.experimental import pallas as pl
from jax.experimental.pallas import tpu as pltpu
from jax.experimental.pallas import tpu_sc as plsc

N = 10000
NP = 10240
E = 320000
D = 128
D_EDGE = 16
HIDDEN = 32

CH = 128
G = CH // 128
NCH = E // CH
NB = 1000

F32 = jnp.float32


E8 = E // 8
BB = 4000


def _alpha_body(ea_ref, w1_ref, b1_ref, w2_ref, s_ref, spl_ref, b2_ref,
                out_ref):
    hi = lax.Precision.HIGHEST
    h1 = jnp.dot(ea_ref[...], w1_ref[...], preferred_element_type=F32,
                 precision=hi) + b1_ref[...]
    h1 = h1 * jax.nn.sigmoid(h1)
    t = h1 * w2_ref[...]
    z8 = jnp.dot(t, s_ref[...], preferred_element_type=F32,
                 precision=hi) + b2_ref[...]
    a8 = jax.nn.sigmoid(z8)
    out_ref[...] = jnp.dot(a8, spl_ref[...], preferred_element_type=F32,
                           precision=hi)


def _alpha16(edge_attr, W1, b1, W2, b2):
    ea_p = edge_attr.reshape(E8, 8 * D_EDGE)
    eye8 = jnp.eye(8, dtype=F32)
    w1big = jnp.einsum("ij,kl->ikjl", eye8, W1).reshape(128, 8 * HIDDEN)
    b1big = jnp.tile(b1, 8).reshape(1, 8 * HIDDEN)
    w2big = jnp.tile(W2[:, 0], 8).reshape(1, 8 * HIDDEN)
    sel = jnp.einsum("ij,k->ikj", eye8, jnp.ones((HIDDEN,), F32))
    sel = sel.reshape(8 * HIDDEN, 8)
    spl = jnp.einsum("ij,k->ijk", eye8, jnp.ones((16,), F32))
    spl = spl.reshape(8, 128)
    b2big = jnp.broadcast_to(b2.reshape(1, 1), (1, 8))
    return pl.pallas_call(
        _alpha_body,
        grid=(E8 // BB,),
        in_specs=[
            pl.BlockSpec((BB, 128), lambda i: (i, 0)),
            pl.BlockSpec((128, 8 * HIDDEN), lambda i: (0, 0)),
            pl.BlockSpec((1, 8 * HIDDEN), lambda i: (0, 0)),
            pl.BlockSpec((1, 8 * HIDDEN), lambda i: (0, 0)),
            pl.BlockSpec((8 * HIDDEN, 8), lambda i: (0, 0)),
            pl.BlockSpec((8, 128), lambda i: (0, 0)),
            pl.BlockSpec((1, 8), lambda i: (0, 0)),
        ],
        out_specs=pl.BlockSpec((BB, 128), lambda i: (i, 0)),
        out_shape=jax.ShapeDtypeStruct((E8, 128), F32),
        compiler_params=pltpu.CompilerParams(
            dimension_semantics=("parallel",)),
    )(ea_p, w1big, b1big, w2big, sel, spl, b2big)


def _issue_scatter(tbl_sh, c_v, rows_v, sem):
    pltpu.async_copy(rows_v, tbl_sh.at[c_v], sem, add=True)


def _wait_scatter(tbl_sh, c_v, rows_v, sem):
    pltpu.make_async_copy(rows_v, tbl_sh.at[c_v], sem).wait()


def _issue_gather(y_hbm, r_v, rows_v, sem):
    pltpu.async_copy(y_hbm.at[r_v], rows_v, sem)


def _wait_gather(y_hbm, r_v, rows_v, sem):
    pltpu.make_async_copy(y_hbm.at[r_v], rows_v, sem).wait()


CHB = 1280
NCHB = E // CHB


def _deg_call(nc, ns):
    nw = nc * ns
    rps = NP // ns

    def body(a16_hbm, col_hbm, out_hbm, col_v, a16_v, deg_v, tmp_v, acc_v,
             bc_v, sh):
        cid = lax.axis_index("c")
        sid = lax.axis_index("s")
        wid = sid * nc + cid
        iot = lax.broadcasted_iota(jnp.int32, (16,), 0)
        zv = jnp.zeros((16,), F32)

        @pl.loop(0, NP, step=16)
        def _(i):
            deg_v[pl.ds(i, 16)] = zv

        @pl.loop(wid, NCHB, step=nw)
        def _(c):
            pltpu.sync_copy(col_hbm.at[pl.ds(c * CHB, CHB)], col_v)
            pltpu.sync_copy(a16_hbm.at[pl.ds(c * (CHB // 8), CHB // 8)],
                            a16_v)

            @pl.loop(0, CHB, step=16)
            def _(i):
                cv = col_v[pl.ds(i, 16)]
                e = i + iot
                av = plsc.load_gather(a16_v, [e >> 3, (e & 7) << 4])
                plsc.addupdate_scatter(deg_v, [cv], av)

        pltpu.sync_copy(deg_v, sh.at[sid])
        plsc.subcore_barrier()

        @pl.loop(0, rps, step=16)
        def _(i):
            acc_v[pl.ds(i, 16)] = zv

        for t in range(16):
            pltpu.sync_copy(sh.at[t, pl.ds(sid * rps, rps)], tmp_v)

            @pl.loop(0, rps, step=16)
            def _(i):
                acc_v[pl.ds(i, 16)] = acc_v[pl.ds(i, 16)] + tmp_v[pl.ds(i, 16)]

        @pl.loop(0, rps)
        def _(i):
            bc_v[i, pl.ds(0, 16)] = plsc.load_gather(
                acc_v, [lax.broadcast(i, (16,))])

        pltpu.sync_copy(bc_v, out_hbm.at[cid, pl.ds(sid * rps, rps)])

    cp = pltpu.CompilerParams()
    if "needs_layout_passes" in pltpu.CompilerParams.__dataclass_fields__:
        cp = dataclasses.replace(cp, needs_layout_passes=False)
    mesh = plsc.VectorSubcoreMesh(core_axis_name="c", subcore_axis_name="s")
    return pl.kernel(
        body,
        out_type=jax.ShapeDtypeStruct((nc, NP, 16), F32),
        mesh=mesh,
        compiler_params=cp,
        scratch_types=[
            pltpu.VMEM((CHB,), jnp.int32),
            pltpu.VMEM((CHB // 8, 128), F32),
            pltpu.VMEM((NP,), F32),
            pltpu.VMEM((NP // ns,), F32),
            pltpu.VMEM((NP // ns,), F32),
            pltpu.VMEM((NP // ns, 16), F32),
            pltpu.VMEM_SHARED((ns, NP), F32),
        ],
    )


def _y_body(x_ref, w_ref, dp_ref, y_ref, dg_ref):
    xw = jnp.dot(x_ref[...], w_ref[...], preferred_element_type=F32,
                 precision=lax.Precision.HIGHEST)
    deg = jnp.sum(dp_ref[...], axis=0)
    dis = jnp.where(deg > 0, lax.rsqrt(deg), 0.0)
    y_ref[...] = xw * dis[:, 0:1]
    dg_ref[...] = deg


def _y_call(nc, x, W, degparts):
    return pl.pallas_call(
        _y_body,
        grid=(N // NB,),
        in_specs=[
            pl.BlockSpec((NB, D), lambda i: (i, 0)),
            pl.BlockSpec((D, D), lambda i: (0, 0)),
            pl.BlockSpec((nc, NB, 16), lambda i: (0, i, 0)),
        ],
        out_specs=[
            pl.BlockSpec((NB, D), lambda i: (i, 0)),
            pl.BlockSpec((NB, 16), lambda i: (i, 0)),
        ],
        out_shape=[
            jax.ShapeDtypeStruct((N, D), F32),
            jax.ShapeDtypeStruct((N, 16), F32),
        ],
        compiler_params=pltpu.CompilerParams(
            dimension_semantics=("parallel",)),
    )(x, W, degparts)


def _agg_call(nc, ns):
    nw = nc * ns
    rps = NP // ns

    def scale(rows_v, a_v):
        @pl.loop(0, CH, step=8)
        def _(e0):
            for j in range(8):
                a = a_v[e0 // 8, pl.ds(j * 16, 16)]
                for k in range(D // 16):
                    sl = pl.ds(k * 16, 16)
                    rows_v[e0 + j, sl] = rows_v[e0 + j, sl] * a

    refs = {}

    def load(c, r_v, c_v, a_v, sem):
        pltpu.async_copy(refs["row"].at[pl.ds(c * CH, CH)], r_v, sem)
        pltpu.async_copy(refs["col"].at[pl.ds(c * CH, CH)], c_v, sem)
        pltpu.async_copy(refs["a16"].at[pl.ds(c * (CH // 8), CH // 8)],
                         a_v, sem)

    def wait_load(r_v, c_v, a_v, sem):
        pltpu.make_async_copy(refs["row"].at[pl.ds(0, CH)], r_v, sem).wait()
        pltpu.make_async_copy(refs["col"].at[pl.ds(0, CH)], c_v, sem).wait()
        pltpu.make_async_copy(refs["a16"].at[pl.ds(0, CH // 8)], a_v,
                              sem).wait()

    def body(y_hbm, a16_hbm, row_hbm, col_hbm, zer_hbm, out_hbm,
             rA, cA, aA, rowsA, rB, cB, aB, rowsB, h_sh,
             slA, slB, sgA, sgB, ssA, ssB):
        refs["row"] = row_hbm
        refs["col"] = col_hbm
        refs["a16"] = a16_hbm
        cid = lax.axis_index("c")
        sid = lax.axis_index("s")
        wid = sid * nc + cid
        nk = (NCH - wid + nw - 1) // nw
        pltpu.sync_copy(zer_hbm, h_sh.at[pl.ds(sid * rps, rps)])
        plsc.subcore_barrier()
        pltpu.sync_copy(row_hbm.at[pl.ds(wid * CH, CH)], rA)
        pltpu.sync_copy(col_hbm.at[pl.ds(wid * CH, CH)], cA)
        pltpu.sync_copy(a16_hbm.at[pl.ds(wid * (CH // 8), CH // 8)], aA)
        _issue_gather(y_hbm, rA, rowsA, sgA)

        @pl.loop(wid, NCH - nw, step=2 * nw)
        def _(c):
            @pl.when(c > wid)
            def _():
                _wait_scatter(h_sh, cB, rowsB, ssB)

            load(c + nw, rB, cB, aB, slB)
            _wait_gather(y_hbm, rA, rowsA, sgA)
            scale(rowsA, aA)
            _issue_scatter(h_sh, cA, rowsA, ssA)
            wait_load(rB, cB, aB, slB)
            _issue_gather(y_hbm, rB, rowsB, sgB)
            _wait_scatter(h_sh, cA, rowsA, ssA)
            cn = lax.select(c + 2 * nw < NCH, c + 2 * nw, c)
            load(cn, rA, cA, aA, slA)
            _wait_gather(y_hbm, rB, rowsB, sgB)
            scale(rowsB, aB)
            _issue_scatter(h_sh, cB, rowsB, ssB)
            wait_load(rA, cA, aA, slA)
            _issue_gather(y_hbm, rA, rowsA, sgA)

        _wait_gather(y_hbm, rA, rowsA, sgA)
        _wait_scatter(h_sh, cB, rowsB, ssB)

        @pl.when(nk % 2 == 1)
        def _():
            scale(rowsA, aA)
            _issue_scatter(h_sh, cA, rowsA, ssA)
            _wait_scatter(h_sh, cA, rowsA, ssA)

        plsc.subcore_barrier()
        pltpu.sync_copy(h_sh.at[pl.ds(sid * rps, rps)],
                        out_hbm.at[cid, pl.ds(sid * rps, rps)])

    mesh = plsc.VectorSubcoreMesh(core_axis_name="c", subcore_axis_name="s")
    return pl.kernel(
        body,
        out_type=jax.ShapeDtypeStruct((nc, NP, D), F32),
        mesh=mesh,
        scratch_types=[
            pltpu.VMEM((CH,), jnp.int32),
            pltpu.VMEM((CH,), jnp.int32),
            pltpu.VMEM((CH // 8, 128), F32),
            pltpu.VMEM((CH, D), F32),
            pltpu.VMEM((CH,), jnp.int32),
            pltpu.VMEM((CH,), jnp.int32),
            pltpu.VMEM((CH // 8, 128), F32),
            pltpu.VMEM((CH, D), F32),
            pltpu.VMEM_SHARED((NP, D), F32),
            pltpu.SemaphoreType.DMA,
            pltpu.SemaphoreType.DMA,
            pltpu.SemaphoreType.DMA,
            pltpu.SemaphoreType.DMA,
            pltpu.SemaphoreType.DMA,
            pltpu.SemaphoreType.DMA,
        ],
    )


def _out_body(hp_ref, dg_ref, x_ref, b_ref, g_ref, be_ref, o_ref):
    agg = jnp.sum(hp_ref[...], axis=0)
    deg = dg_ref[...]
    dis = jnp.where(deg > 0, lax.rsqrt(deg), 0.0)[:, 0:1]
    h = agg * dis + b_ref[...]
    mu = jnp.mean(h, axis=-1, keepdims=True)
    var = jnp.mean((h - mu) ** 2, axis=-1, keepdims=True)
    h = (h - mu) * lax.rsqrt(var + 1e-5) * g_ref[...] + be_ref[...]
    h = h * jax.nn.sigmoid(h)
    o_ref[...] = h + x_ref[...]


def _out_call(nc, hparts, degsum, x, b, gamma, beta):
    return pl.pallas_call(
        _out_body,
        grid=(N // NB,),
        in_specs=[
            pl.BlockSpec((nc, NB, D), lambda i: (0, i, 0)),
            pl.BlockSpec((NB, 16), lambda i: (i, 0)),
            pl.BlockSpec((NB, D), lambda i: (i, 0)),
            pl.BlockSpec((1, D), lambda i: (0, 0)),
            pl.BlockSpec((1, D), lambda i: (0, 0)),
            pl.BlockSpec((1, D), lambda i: (0, 0)),
        ],
        out_specs=pl.BlockSpec((NB, D), lambda i: (i, 0)),
        out_shape=jax.ShapeDtypeStruct((N, D), F32),
        compiler_params=pltpu.CompilerParams(
            dimension_semantics=("parallel",)),
    )(hparts, degsum, x, b.reshape(1, D), gamma.reshape(1, D),
      beta.reshape(1, D))


def kernel(x, edge_index, edge_attr, W, b, W1, b1, W2, b2, gamma, beta):
    info = plsc.get_sparse_core_info()
    nc, ns = info.num_cores, info.num_subcores

    row = edge_index[0]
    col = edge_index[1]

    alpha16 = _alpha16(edge_attr, W1, b1, W2, b2)

    zeros128 = jnp.zeros((NP // ns, D), F32)

    degparts = _deg_call(nc, ns)(alpha16, col)
    y, degsum = _y_call(nc, x, W, degparts)
    hparts = _agg_call(nc, ns)(y, alpha16, row, col, zeros128)
    return _out_call(nc, hparts, degsum, x, b, gamma, beta)


# speedup vs baseline: 10.7371x; 1.0009x over previous
import dataclasses
import functools

import jax
import jax.numpy as jnp
from jax import lax
from jax.experimental import pallas as pl
from jax.experimental.pallas import tpu as pltpu
from jax.experimental.pallas import tpu_sc as plsc

N = 10000
NP = 10240
E = 320000
D = 128
D_EDGE = 16
HIDDEN = 32

CH = 128
G = CH // 128
NCH = E // CH
NB = 1000

F32 = jnp.float32


E8 = E // 8
BB = 4000


def _alpha_body(ea_ref, w1_ref, b1_ref, w2_ref, s_ref, spl_ref, b2_ref,
                out_ref):
    hi = lax.Precision.HIGHEST
    h1 = jnp.dot(ea_ref[...], w1_ref[...], preferred_element_type=F32,
                 precision=hi) + b1_ref[...]
    h1 = h1 * jax.nn.sigmoid(h1)
    t = h1 * w2_ref[...]
    z8 = jnp.dot(t, s_ref[...], preferred_element_type=F32,
                 precision=hi) + b2_ref[...]
    a8 = jax.nn.sigmoid(z8)
    out_ref[...] = jnp.dot(a8, spl_ref[...], preferred_element_type=F32,
                           precision=hi)


def _alpha16(edge_attr, W1, b1, W2, b2):
    ea_p = edge_attr.reshape(E8, 8 * D_EDGE)
    eye8 = jnp.eye(8, dtype=F32)
    w1big = jnp.einsum("ij,kl->ikjl", eye8, W1).reshape(128, 8 * HIDDEN)
    b1big = jnp.tile(b1, 8).reshape(1, 8 * HIDDEN)
    w2big = jnp.tile(W2[:, 0], 8).reshape(1, 8 * HIDDEN)
    sel = jnp.einsum("ij,k->ikj", eye8, jnp.ones((HIDDEN,), F32))
    sel = sel.reshape(8 * HIDDEN, 8)
    spl = jnp.einsum("ij,k->ijk", eye8, jnp.ones((16,), F32))
    spl = spl.reshape(8, 128)
    b2big = jnp.broadcast_to(b2.reshape(1, 1), (1, 8))
    return pl.pallas_call(
        _alpha_body,
        grid=(E8 // BB,),
        in_specs=[
            pl.BlockSpec((BB, 128), lambda i: (i, 0)),
            pl.BlockSpec((128, 8 * HIDDEN), lambda i: (0, 0)),
            pl.BlockSpec((1, 8 * HIDDEN), lambda i: (0, 0)),
            pl.BlockSpec((1, 8 * HIDDEN), lambda i: (0, 0)),
            pl.BlockSpec((8 * HIDDEN, 8), lambda i: (0, 0)),
            pl.BlockSpec((8, 128), lambda i: (0, 0)),
            pl.BlockSpec((1, 8), lambda i: (0, 0)),
        ],
        out_specs=pl.BlockSpec((BB, 128), lambda i: (i, 0)),
        out_shape=jax.ShapeDtypeStruct((E8, 128), F32),
        compiler_params=pltpu.CompilerParams(
            dimension_semantics=("parallel",)),
    )(ea_p, w1big, b1big, w2big, sel, spl, b2big)


def _issue_scatter(tbl_sh, c_v, rows_v, sem):
    pltpu.async_copy(rows_v, tbl_sh.at[c_v], sem, add=True)


def _wait_scatter(tbl_sh, c_v, rows_v, sem):
    pltpu.make_async_copy(rows_v, tbl_sh.at[c_v], sem).wait()


def _issue_gather(y_hbm, r_v, rows_v, sem):
    pltpu.async_copy(y_hbm.at[r_v], rows_v, sem)


def _wait_gather(y_hbm, r_v, rows_v, sem):
    pltpu.make_async_copy(y_hbm.at[r_v], rows_v, sem).wait()


CHB = 1280
NCHB = E // CHB


def _deg_call(nc, ns):
    nw = nc * ns
    rps = NP // ns

    def body(a16_hbm, col_hbm, out_hbm, col_v, a16_v, deg_v, tmp_v, acc_v,
             bc_v, sh):
        cid = lax.axis_index("c")
        sid = lax.axis_index("s")
        wid = sid * nc + cid
        iot = lax.broadcasted_iota(jnp.int32, (16,), 0)
        zv = jnp.zeros((16,), F32)

        @pl.loop(0, NP, step=16)
        def _(i):
            deg_v[pl.ds(i, 16)] = zv

        @pl.loop(wid, NCHB, step=nw)
        def _(c):
            pltpu.sync_copy(col_hbm.at[pl.ds(c * CHB, CHB)], col_v)
            pltpu.sync_copy(a16_hbm.at[pl.ds(c * (CHB // 8), CHB // 8)],
                            a16_v)

            @pl.loop(0, CHB, step=16)
            def _(i):
                cv = col_v[pl.ds(i, 16)]
                e = i + iot
                av = plsc.load_gather(a16_v, [e >> 3, (e & 7) << 4])
                plsc.addupdate_scatter(deg_v, [cv], av)

        pltpu.sync_copy(deg_v, sh.at[sid])
        plsc.subcore_barrier()

        @pl.loop(0, rps, step=16)
        def _(i):
            acc_v[pl.ds(i, 16)] = zv

        for t in range(16):
            pltpu.sync_copy(sh.at[t, pl.ds(sid * rps, rps)], tmp_v)

            @pl.loop(0, rps, step=16)
            def _(i):
                acc_v[pl.ds(i, 16)] = acc_v[pl.ds(i, 16)] + tmp_v[pl.ds(i, 16)]

        @pl.loop(0, rps)
        def _(i):
            bc_v[i, pl.ds(0, 16)] = plsc.load_gather(
                acc_v, [lax.broadcast(i, (16,))])

        pltpu.sync_copy(bc_v, out_hbm.at[cid, pl.ds(sid * rps, rps)])

    cp = pltpu.CompilerParams()
    if "needs_layout_passes" in pltpu.CompilerParams.__dataclass_fields__:
        cp = dataclasses.replace(cp, needs_layout_passes=False)
    mesh = plsc.VectorSubcoreMesh(core_axis_name="c", subcore_axis_name="s")
    return pl.kernel(
        body,
        out_type=jax.ShapeDtypeStruct((nc, NP, 16), F32),
        mesh=mesh,
        compiler_params=cp,
        scratch_types=[
            pltpu.VMEM((CHB,), jnp.int32),
            pltpu.VMEM((CHB // 8, 128), F32),
            pltpu.VMEM((NP,), F32),
            pltpu.VMEM((NP // ns,), F32),
            pltpu.VMEM((NP // ns,), F32),
            pltpu.VMEM((NP // ns, 16), F32),
            pltpu.VMEM_SHARED((ns, NP), F32),
        ],
    )


def _y_body(x_ref, w_ref, dp_ref, y_ref, dg_ref):
    xw = jnp.dot(x_ref[...], w_ref[...], preferred_element_type=F32,
                 precision=lax.Precision.HIGHEST)
    deg = jnp.sum(dp_ref[...], axis=0)
    dis = jnp.where(deg > 0, lax.rsqrt(deg), 0.0)
    y_ref[...] = xw * dis[:, 0:1]
    dg_ref[...] = deg


def _y_call(nc, x, W, degparts):
    return pl.pallas_call(
        _y_body,
        grid=(N // NB,),
        in_specs=[
            pl.BlockSpec((NB, D), lambda i: (i, 0)),
            pl.BlockSpec((D, D), lambda i: (0, 0)),
            pl.BlockSpec((nc, NB, 16), lambda i: (0, i, 0)),
        ],
        out_specs=[
            pl.BlockSpec((NB, D), lambda i: (i, 0)),
            pl.BlockSpec((NB, 16), lambda i: (i, 0)),
        ],
        out_shape=[
            jax.ShapeDtypeStruct((N, D), F32),
            jax.ShapeDtypeStruct((N, 16), F32),
        ],
        compiler_params=pltpu.CompilerParams(
            dimension_semantics=("parallel",)),
    )(x, W, degparts)


def _agg_call(nc, ns):
    nw = nc * ns
    rps = NP // ns

    def scale(rows_v, a_v):
        @pl.loop(0, CH, step=8)
        def _(e0):
            for j in range(8):
                a = a_v[e0 // 8, pl.ds(j * 16, 16)]
                vs = [rows_v[e0 + j, pl.ds(k * 16, 16)] * a
                      for k in range(D // 16)]
                for k in range(D // 16):
                    rows_v[e0 + j, pl.ds(k * 16, 16)] = vs[k]

    refs = {}

    def load(c, r_v, c_v, a_v, sem):
        pltpu.async_copy(refs["row"].at[pl.ds(c * CH, CH)], r_v, sem)
        pltpu.async_copy(refs["col"].at[pl.ds(c * CH, CH)], c_v, sem)
        pltpu.async_copy(refs["a16"].at[pl.ds(c * (CH // 8), CH // 8)],
                         a_v, sem)

    def wait_load(r_v, c_v, a_v, sem):
        pltpu.make_async_copy(refs["row"].at[pl.ds(0, CH)], r_v, sem).wait()
        pltpu.make_async_copy(refs["col"].at[pl.ds(0, CH)], c_v, sem).wait()
        pltpu.make_async_copy(refs["a16"].at[pl.ds(0, CH // 8)], a_v,
                              sem).wait()

    def body(y_hbm, a16_hbm, row_hbm, col_hbm, zer_hbm, out_hbm,
             rA, cA, aA, rowsA, rB, cB, aB, rowsB, h_sh,
             slA, slB, sgA, sgB, ssA, ssB):
        refs["row"] = row_hbm
        refs["col"] = col_hbm
        refs["a16"] = a16_hbm
        cid = lax.axis_index("c")
        sid = lax.axis_index("s")
        wid = sid * nc + cid
        nk = (NCH - wid + nw - 1) // nw
        pltpu.sync_copy(zer_hbm, h_sh.at[pl.ds(sid * rps, rps)])
        plsc.subcore_barrier()
        pltpu.sync_copy(row_hbm.at[pl.ds(wid * CH, CH)], rA)
        pltpu.sync_copy(col_hbm.at[pl.ds(wid * CH, CH)], cA)
        pltpu.sync_copy(a16_hbm.at[pl.ds(wid * (CH // 8), CH // 8)], aA)
        _issue_gather(y_hbm, rA, rowsA, sgA)

        @pl.loop(wid, NCH - nw, step=2 * nw)
        def _(c):
            @pl.when(c > wid)
            def _():
                _wait_scatter(h_sh, cB, rowsB, ssB)

            load(c + nw, rB, cB, aB, slB)
            _wait_gather(y_hbm, rA, rowsA, sgA)
            scale(rowsA, aA)
            _issue_scatter(h_sh, cA, rowsA, ssA)
            wait_load(rB, cB, aB, slB)
            _issue_gather(y_hbm, rB, rowsB, sgB)
            _wait_scatter(h_sh, cA, rowsA, ssA)
            cn = lax.select(c + 2 * nw < NCH, c + 2 * nw, c)
            load(cn, rA, cA, aA, slA)
            _wait_gather(y_hbm, rB, rowsB, sgB)
            scale(rowsB, aB)
            _issue_scatter(h_sh, cB, rowsB, ssB)
            wait_load(rA, cA, aA, slA)
            _issue_gather(y_hbm, rA, rowsA, sgA)

        _wait_gather(y_hbm, rA, rowsA, sgA)
        _wait_scatter(h_sh, cB, rowsB, ssB)

        @pl.when(nk % 2 == 1)
        def _():
            scale(rowsA, aA)
            _issue_scatter(h_sh, cA, rowsA, ssA)
            _wait_scatter(h_sh, cA, rowsA, ssA)

        plsc.subcore_barrier()
        pltpu.sync_copy(h_sh.at[pl.ds(sid * rps, rps)],
                        out_hbm.at[cid, pl.ds(sid * rps, rps)])

    mesh = plsc.VectorSubcoreMesh(core_axis_name="c", subcore_axis_name="s")
    return pl.kernel(
        body,
        out_type=jax.ShapeDtypeStruct((nc, NP, D), F32),
        mesh=mesh,
        scratch_types=[
            pltpu.VMEM((CH,), jnp.int32),
            pltpu.VMEM((CH,), jnp.int32),
            pltpu.VMEM((CH // 8, 128), F32),
            pltpu.VMEM((CH, D), F32),
            pltpu.VMEM((CH,), jnp.int32),
            pltpu.VMEM((CH,), jnp.int32),
            pltpu.VMEM((CH // 8, 128), F32),
            pltpu.VMEM((CH, D), F32),
            pltpu.VMEM_SHARED((NP, D), F32),
            pltpu.SemaphoreType.DMA,
            pltpu.SemaphoreType.DMA,
            pltpu.SemaphoreType.DMA,
            pltpu.SemaphoreType.DMA,
            pltpu.SemaphoreType.DMA,
            pltpu.SemaphoreType.DMA,
        ],
    )


def _out_body(hp_ref, dg_ref, x_ref, b_ref, g_ref, be_ref, o_ref):
    agg = jnp.sum(hp_ref[...], axis=0)
    deg = dg_ref[...]
    dis = jnp.where(deg > 0, lax.rsqrt(deg), 0.0)[:, 0:1]
    h = agg * dis + b_ref[...]
    mu = jnp.mean(h, axis=-1, keepdims=True)
    var = jnp.mean((h - mu) ** 2, axis=-1, keepdims=True)
    h = (h - mu) * lax.rsqrt(var + 1e-5) * g_ref[...] + be_ref[...]
    h = h * jax.nn.sigmoid(h)
    o_ref[...] = h + x_ref[...]


def _out_call(nc, hparts, degsum, x, b, gamma, beta):
    return pl.pallas_call(
        _out_body,
        grid=(N // NB,),
        in_specs=[
            pl.BlockSpec((nc, NB, D), lambda i: (0, i, 0)),
            pl.BlockSpec((NB, 16), lambda i: (i, 0)),
            pl.BlockSpec((NB, D), lambda i: (i, 0)),
            pl.BlockSpec((1, D), lambda i: (0, 0)),
            pl.BlockSpec((1, D), lambda i: (0, 0)),
            pl.BlockSpec((1, D), lambda i: (0, 0)),
        ],
        out_specs=pl.BlockSpec((NB, D), lambda i: (i, 0)),
        out_shape=jax.ShapeDtypeStruct((N, D), F32),
        compiler_params=pltpu.CompilerParams(
            dimension_semantics=("parallel",)),
    )(hparts, degsum, x, b.reshape(1, D), gamma.reshape(1, D),
      beta.reshape(1, D))


def kernel(x, edge_index, edge_attr, W, b, W1, b1, W2, b2, gamma, beta):
    info = plsc.get_sparse_core_info()
    nc, ns = info.num_cores, info.num_subcores

    row = edge_index[0]
    col = edge_index[1]

    alpha16 = _alpha16(edge_attr, W1, b1, W2, b2)

    zeros128 = jnp.zeros((NP // ns, D), F32)

    degparts = _deg_call(nc, ns)(alpha16, col)
    y, degsum = _y_call(nc, x, W, degparts)
    hparts = _agg_call(nc, ns)(y, alpha16, row, col, zeros128)
    return _out_call(nc, hparts, degsum, x, b, gamma, beta)


# speedup vs baseline: 12.5835x; 1.1720x over previous
import dataclasses
import functools

import jax
import jax.numpy as jnp
from jax import lax
from jax.experimental import pallas as pl
from jax.experimental.pallas import tpu as pltpu
from jax.experimental.pallas import tpu_sc as plsc

N = 10000
NP = 10240
E = 320000
D = 128
D_EDGE = 16
HIDDEN = 32

CH = 128
G = CH // 128
NCH = E // CH
NB = 1000

F32 = jnp.float32


E8 = E // 8
BB = 4000


def _alpha_body(ea_ref, w1_ref, b1_ref, w2_ref, s_ref, spl_ref, b2_ref,
                out_ref):
    hi = lax.Precision.HIGHEST
    h1 = jnp.dot(ea_ref[...], w1_ref[...], preferred_element_type=F32,
                 precision=hi) + b1_ref[...]
    h1 = h1 * jax.nn.sigmoid(h1)
    t = h1 * w2_ref[...]
    z8 = jnp.dot(t, s_ref[...], preferred_element_type=F32) + b2_ref[...]
    a8 = jax.nn.sigmoid(z8)
    out_ref[...] = jnp.dot(a8, spl_ref[...],
                           preferred_element_type=F32)


def _alpha16(edge_attr, W1, b1, W2, b2):
    ea_p = edge_attr.reshape(E8, 8 * D_EDGE)
    eye8 = jnp.eye(8, dtype=F32)
    w1big = jnp.einsum("ij,kl->ikjl", eye8, W1).reshape(128, 8 * HIDDEN)
    b1big = jnp.tile(b1, 8).reshape(1, 8 * HIDDEN)
    w2big = jnp.tile(W2[:, 0], 8).reshape(1, 8 * HIDDEN)
    sel = jnp.einsum("ij,k->ikj", eye8, jnp.ones((HIDDEN,), F32))
    sel = sel.reshape(8 * HIDDEN, 8)
    spl = jnp.einsum("ij,k->ijk", eye8, jnp.ones((16,), F32))
    spl = spl.reshape(8, 128)
    b2big = jnp.broadcast_to(b2.reshape(1, 1), (1, 8))
    return pl.pallas_call(
        _alpha_body,
        grid=(E8 // BB,),
        in_specs=[
            pl.BlockSpec((BB, 128), lambda i: (i, 0)),
            pl.BlockSpec((128, 8 * HIDDEN), lambda i: (0, 0)),
            pl.BlockSpec((1, 8 * HIDDEN), lambda i: (0, 0)),
            pl.BlockSpec((1, 8 * HIDDEN), lambda i: (0, 0)),
            pl.BlockSpec((8 * HIDDEN, 8), lambda i: (0, 0)),
            pl.BlockSpec((8, 128), lambda i: (0, 0)),
            pl.BlockSpec((1, 8), lambda i: (0, 0)),
        ],
        out_specs=pl.BlockSpec((BB, 128), lambda i: (i, 0)),
        out_shape=jax.ShapeDtypeStruct((E8, 128), F32),
        compiler_params=pltpu.CompilerParams(
            dimension_semantics=("parallel",)),
    )(ea_p, w1big, b1big, w2big, sel, spl, b2big)


def _issue_scatter(tbl_sh, c_v, rows_v, sem):
    pltpu.async_copy(rows_v, tbl_sh.at[c_v], sem, add=True)


def _wait_scatter(tbl_sh, c_v, rows_v, sem):
    pltpu.make_async_copy(rows_v, tbl_sh.at[c_v], sem).wait()


def _issue_gather(y_hbm, r_v, rows_v, sem):
    pltpu.async_copy(y_hbm.at[r_v], rows_v, sem)


def _wait_gather(y_hbm, r_v, rows_v, sem):
    pltpu.make_async_copy(y_hbm.at[r_v], rows_v, sem).wait()


CHB = 1280
NCHB = E // CHB


def _deg_call(nc, ns):
    nw = nc * ns
    rps = NP // ns

    def body(a16_hbm, col_hbm, out_hbm, col_v, a16_v, deg_v, tmp_v, acc_v,
             bc_v, sh):
        cid = lax.axis_index("c")
        sid = lax.axis_index("s")
        wid = sid * nc + cid
        iot = lax.broadcasted_iota(jnp.int32, (16,), 0)
        zv = jnp.zeros((16,), F32)

        @pl.loop(0, NP, step=16)
        def _(i):
            deg_v[pl.ds(i, 16)] = zv

        @pl.loop(wid, NCHB, step=nw)
        def _(c):
            pltpu.sync_copy(col_hbm.at[pl.ds(c * CHB, CHB)], col_v)
            pltpu.sync_copy(a16_hbm.at[pl.ds(c * (CHB // 8), CHB // 8)],
                            a16_v)

            @pl.loop(0, CHB, step=16)
            def _(i):
                cv = col_v[pl.ds(i, 16)]
                e = i + iot
                av = plsc.load_gather(a16_v, [e >> 3, (e & 7) << 4])
                plsc.addupdate_scatter(deg_v, [cv], av)

        pltpu.sync_copy(deg_v, sh.at[sid])
        plsc.subcore_barrier()

        @pl.loop(0, rps, step=16)
        def _(i):
            acc_v[pl.ds(i, 16)] = zv

        for t in range(16):
            pltpu.sync_copy(sh.at[t, pl.ds(sid * rps, rps)], tmp_v)

            @pl.loop(0, rps, step=16)
            def _(i):
                acc_v[pl.ds(i, 16)] = acc_v[pl.ds(i, 16)] + tmp_v[pl.ds(i, 16)]

        @pl.loop(0, rps)
        def _(i):
            bc_v[i, pl.ds(0, 16)] = plsc.load_gather(
                acc_v, [lax.broadcast(i, (16,))])

        pltpu.sync_copy(bc_v, out_hbm.at[cid, pl.ds(sid * rps, rps)])

    cp = pltpu.CompilerParams()
    if "needs_layout_passes" in pltpu.CompilerParams.__dataclass_fields__:
        cp = dataclasses.replace(cp, needs_layout_passes=False)
    mesh = plsc.VectorSubcoreMesh(core_axis_name="c", subcore_axis_name="s")
    return pl.kernel(
        body,
        out_type=jax.ShapeDtypeStruct((nc, NP, 16), F32),
        mesh=mesh,
        compiler_params=cp,
        scratch_types=[
            pltpu.VMEM((CHB,), jnp.int32),
            pltpu.VMEM((CHB // 8, 128), F32),
            pltpu.VMEM((NP,), F32),
            pltpu.VMEM((NP // ns,), F32),
            pltpu.VMEM((NP // ns,), F32),
            pltpu.VMEM((NP // ns, 16), F32),
            pltpu.VMEM_SHARED((ns, NP), F32),
        ],
    )


def _y_body(x_ref, w_ref, dp_ref, y_ref, dg_ref):
    xw = jnp.dot(x_ref[...], w_ref[...], preferred_element_type=F32,
                 precision=lax.Precision.HIGHEST)
    deg = jnp.sum(dp_ref[...], axis=0)
    dis = jnp.where(deg > 0, lax.rsqrt(deg), 0.0)
    y_ref[...] = xw * dis[:, 0:1]
    dg_ref[...] = deg


def _y_call(nc, x, W, degparts):
    return pl.pallas_call(
        _y_body,
        grid=(N // NB,),
        in_specs=[
            pl.BlockSpec((NB, D), lambda i: (i, 0)),
            pl.BlockSpec((D, D), lambda i: (0, 0)),
            pl.BlockSpec((nc, NB, 16), lambda i: (0, i, 0)),
        ],
        out_specs=[
            pl.BlockSpec((NB, D), lambda i: (i, 0)),
            pl.BlockSpec((NB, 16), lambda i: (i, 0)),
        ],
        out_shape=[
            jax.ShapeDtypeStruct((N, D), F32),
            jax.ShapeDtypeStruct((N, 16), F32),
        ],
        compiler_params=pltpu.CompilerParams(
            dimension_semantics=("parallel",)),
    )(x, W, degparts)


def _agg_call(nc, ns):
    nw = nc * ns
    rps = NP // ns

    def scale(rows_v, a_v):
        @pl.loop(0, CH, step=8)
        def _(e0):
            for j in range(8):
                a = a_v[e0 // 8, pl.ds(j * 16, 16)]
                vs = [rows_v[e0 + j, pl.ds(k * 16, 16)] * a
                      for k in range(D // 16)]
                for k in range(D // 16):
                    rows_v[e0 + j, pl.ds(k * 16, 16)] = vs[k]

    refs = {}

    def load(c, r_v, c_v, a_v, sem):
        pltpu.async_copy(refs["row"].at[pl.ds(c * CH, CH)], r_v, sem)
        pltpu.async_copy(refs["col"].at[pl.ds(c * CH, CH)], c_v, sem)
        pltpu.async_copy(refs["a16"].at[pl.ds(c * (CH // 8), CH // 8)],
                         a_v, sem)

    def wait_load(r_v, c_v, a_v, sem):
        pltpu.make_async_copy(refs["row"].at[pl.ds(0, CH)], r_v, sem).wait()
        pltpu.make_async_copy(refs["col"].at[pl.ds(0, CH)], c_v, sem).wait()
        pltpu.make_async_copy(refs["a16"].at[pl.ds(0, CH // 8)], a_v,
                              sem).wait()

    def body(y_hbm, a16_hbm, row_hbm, col_hbm, zer_hbm, out_hbm,
             rA, cA, aA, rowsA, rB, cB, aB, rowsB, h_sh,
             slA, slB, sgA, sgB, ssA, ssB):
        refs["row"] = row_hbm
        refs["col"] = col_hbm
        refs["a16"] = a16_hbm
        cid = lax.axis_index("c")
        sid = lax.axis_index("s")
        wid = sid * nc + cid
        nk = (NCH - wid + nw - 1) // nw
        pltpu.sync_copy(zer_hbm, h_sh.at[pl.ds(sid * rps, rps)])
        plsc.subcore_barrier()
        pltpu.sync_copy(row_hbm.at[pl.ds(wid * CH, CH)], rA)
        pltpu.sync_copy(col_hbm.at[pl.ds(wid * CH, CH)], cA)
        pltpu.sync_copy(a16_hbm.at[pl.ds(wid * (CH // 8), CH // 8)], aA)
        _issue_gather(y_hbm, rA, rowsA, sgA)

        @pl.loop(wid, NCH - nw, step=2 * nw)
        def _(c):
            @pl.when(c > wid)
            def _():
                _wait_scatter(h_sh, cB, rowsB, ssB)

            load(c + nw, rB, cB, aB, slB)
            _wait_gather(y_hbm, rA, rowsA, sgA)
            scale(rowsA, aA)
            _issue_scatter(h_sh, cA, rowsA, ssA)
            wait_load(rB, cB, aB, slB)
            _issue_gather(y_hbm, rB, rowsB, sgB)
            _wait_scatter(h_sh, cA, rowsA, ssA)
            cn = lax.select(c + 2 * nw < NCH, c + 2 * nw, c)
            load(cn, rA, cA, aA, slA)
            _wait_gather(y_hbm, rB, rowsB, sgB)
            scale(rowsB, aB)
            _issue_scatter(h_sh, cB, rowsB, ssB)
            wait_load(rA, cA, aA, slA)
            _issue_gather(y_hbm, rA, rowsA, sgA)

        _wait_gather(y_hbm, rA, rowsA, sgA)
        _wait_scatter(h_sh, cB, rowsB, ssB)

        @pl.when(nk % 2 == 1)
        def _():
            scale(rowsA, aA)
            _issue_scatter(h_sh, cA, rowsA, ssA)
            _wait_scatter(h_sh, cA, rowsA, ssA)

        plsc.subcore_barrier()
        pltpu.sync_copy(h_sh.at[pl.ds(sid * rps, rps)],
                        out_hbm.at[cid, pl.ds(sid * rps, rps)])

    mesh = plsc.VectorSubcoreMesh(core_axis_name="c", subcore_axis_name="s")
    return pl.kernel(
        body,
        out_type=jax.ShapeDtypeStruct((nc, NP, D), F32),
        mesh=mesh,
        scratch_types=[
            pltpu.VMEM((CH,), jnp.int32),
            pltpu.VMEM((CH,), jnp.int32),
            pltpu.VMEM((CH // 8, 128), F32),
            pltpu.VMEM((CH, D), F32),
            pltpu.VMEM((CH,), jnp.int32),
            pltpu.VMEM((CH,), jnp.int32),
            pltpu.VMEM((CH // 8, 128), F32),
            pltpu.VMEM((CH, D), F32),
            pltpu.VMEM_SHARED((NP, D), F32),
            pltpu.SemaphoreType.DMA,
            pltpu.SemaphoreType.DMA,
            pltpu.SemaphoreType.DMA,
            pltpu.SemaphoreType.DMA,
            pltpu.SemaphoreType.DMA,
            pltpu.SemaphoreType.DMA,
        ],
    )


def _out_body(hp_ref, dg_ref, x_ref, b_ref, g_ref, be_ref, o_ref):
    agg = jnp.sum(hp_ref[...], axis=0)
    deg = dg_ref[...]
    dis = jnp.where(deg > 0, lax.rsqrt(deg), 0.0)[:, 0:1]
    h = agg * dis + b_ref[...]
    mu = jnp.mean(h, axis=-1, keepdims=True)
    var = jnp.mean((h - mu) ** 2, axis=-1, keepdims=True)
    h = (h - mu) * lax.rsqrt(var + 1e-5) * g_ref[...] + be_ref[...]
    h = h * jax.nn.sigmoid(h)
    o_ref[...] = h + x_ref[...]


def _out_call(nc, hparts, degsum, x, b, gamma, beta):
    return pl.pallas_call(
        _out_body,
        grid=(N // NB,),
        in_specs=[
            pl.BlockSpec((nc, NB, D), lambda i: (0, i, 0)),
            pl.BlockSpec((NB, 16), lambda i: (i, 0)),
            pl.BlockSpec((NB, D), lambda i: (i, 0)),
            pl.BlockSpec((1, D), lambda i: (0, 0)),
            pl.BlockSpec((1, D), lambda i: (0, 0)),
            pl.BlockSpec((1, D), lambda i: (0, 0)),
        ],
        out_specs=pl.BlockSpec((NB, D), lambda i: (i, 0)),
        out_shape=jax.ShapeDtypeStruct((N, D), F32),
        compiler_params=pltpu.CompilerParams(
            dimension_semantics=("parallel",)),
    )(hparts, degsum, x, b.reshape(1, D), gamma.reshape(1, D),
      beta.reshape(1, D))


def kernel(x, edge_index, edge_attr, W, b, W1, b1, W2, b2, gamma, beta):
    info = plsc.get_sparse_core_info()
    nc, ns = info.num_cores, info.num_subcores

    row = edge_index[0]
    col = edge_index[1]

    alpha16 = _alpha16(edge_attr, W1, b1, W2, b2)

    zeros128 = jnp.zeros((NP // ns, D), F32)

    degparts = _deg_call(nc, ns)(alpha16, col)
    y, degsum = _y_call(nc, x, W, degparts)
    hparts = _agg_call(nc, ns)(y, alpha16, row, col, zeros128)
    return _out_call(nc, hparts, degsum, x, b, gamma, beta)
